# Initial kernel scaffold; baseline (speedup 1.0000x reference)
import jax
import jax.numpy as jnp
from jax.experimental import pallas as pl


def kernel(h, x, edge_index, edge_attr, t, mask, params):
    raise NotImplementedError("write your pallas kernel here")


# speedup vs baseline: 2.1467x; 2.1467x over previous
import functools

import jax
import jax.numpy as jnp
from jax import lax
from jax.experimental import pallas as pl
from jax.experimental.pallas import tpu as pltpu
from jax.experimental.pallas import tpu_sc as plsc

F32 = jnp.float32

NC = 2
NS = 16
NW = NC * NS

CH = 80


def _silu(x):
    return x * jax.nn.sigmoid(x)


def _dot(a, b):
    return jnp.dot(a, b, preferred_element_type=F32)


def _prep_call(t, wt1, bt1, wt2, bt2, we, w1d_stack, be, b1_stack, num_layers):
    def body(t_ref, wt1_ref, bt1_ref, wt2_ref, bt2_ref, we_ref, w1d_ref,
             be_ref, b1_ref, te_ref, m_ref, beff_ref):
        tval = t_ref[0]
        u = _silu(tval * wt1_ref[...] + bt1_ref[...])
        te_ref[...] = _dot(u, wt2_ref[...]) + bt2_ref[...]
        for l in range(num_layers):
            w1d = w1d_ref[l]
            m_ref[l] = _dot(we_ref[...], w1d)
            beff_ref[l] = _dot(be_ref[...], w1d) + b1_ref[l]

    H = wt2.shape[0]
    return pl.pallas_call(
        body,
        in_specs=[
            pl.BlockSpec(memory_space=pltpu.SMEM),
            pl.BlockSpec((1, H), lambda: (0, 0)),
            pl.BlockSpec((1, H), lambda: (0, 0)),
            pl.BlockSpec((H, H), lambda: (0, 0)),
            pl.BlockSpec((1, H), lambda: (0, 0)),
            pl.BlockSpec((16, H), lambda: (0, 0)),
            pl.BlockSpec((num_layers, H, H), lambda: (0, 0, 0)),
            pl.BlockSpec((1, H), lambda: (0, 0)),
            pl.BlockSpec((num_layers, 1, H), lambda: (0, 0, 0)),
        ],
        out_specs=[
            pl.BlockSpec((1, H), lambda: (0, 0)),
            pl.BlockSpec((num_layers, 16, H), lambda: (0, 0, 0)),
            pl.BlockSpec((num_layers, 1, H), lambda: (0, 0, 0)),
        ],
        out_shape=[
            jax.ShapeDtypeStruct((1, H), F32),
            jax.ShapeDtypeStruct((num_layers, 16, H), F32),
            jax.ShapeDtypeStruct((num_layers, 1, H), F32),
        ],
    )(t, wt1, bt1, wt2, bt2, we, w1d_stack, be, b1_stack)


def _embed_call(h, wn, bn, te, blk):
    n, din = h.shape
    H = wn.shape[1]

    def body(h_ref, wn_ref, bn_ref, te_ref, out_ref):
        out_ref[...] = (_dot(h_ref[...], wn_ref[...]) + bn_ref[...]
                        + te_ref[...])

    return pl.pallas_call(
        body,
        grid=(n // blk,),
        in_specs=[
            pl.BlockSpec((blk, din), lambda i: (i, 0)),
            pl.BlockSpec((din, H), lambda i: (0, 0)),
            pl.BlockSpec((1, H), lambda i: (0, 0)),
            pl.BlockSpec((1, H), lambda i: (0, 0)),
        ],
        out_specs=pl.BlockSpec((blk, H), lambda i: (i, 0)),
        out_shape=jax.ShapeDtypeStruct((n, H), F32),
    )(h, wn, bn, te)


def _proj_call(hf, w1a, w1b, blk):
    n, H = hf.shape

    def body(h_ref, wa_ref, wb_ref, p_ref, q_ref):
        hv = h_ref[...]
        p_ref[...] = _dot(hv, wa_ref[...])
        q_ref[...] = _dot(hv, wb_ref[...])

    return pl.pallas_call(
        body,
        grid=(n // blk,),
        in_specs=[
            pl.BlockSpec((blk, H), lambda i: (i, 0)),
            pl.BlockSpec((H, H), lambda i: (0, 0)),
            pl.BlockSpec((H, H), lambda i: (0, 0)),
        ],
        out_specs=[
            pl.BlockSpec((blk, H), lambda i: (i, 0)),
            pl.BlockSpec((blk, H), lambda i: (i, 0)),
        ],
        out_shape=[
            jax.ShapeDtypeStruct((n, H), F32),
            jax.ShapeDtypeStruct((n, H), F32),
        ],
    )(hf, w1a, w1b)


def _sc_gather_call(P, Q, X, row, col):
    n, H = P.shape
    e = row.shape[0]
    e_per_w = e // NW
    n_sub = e_per_w // CH
    mesh = plsc.VectorSubcoreMesh(core_axis_name="c", subcore_axis_name="s")

    @functools.partial(
        pl.kernel,
        out_type=(
            jax.ShapeDtypeStruct((e, H), F32),
            jax.ShapeDtypeStruct((e, H), F32),
            jax.ShapeDtypeStruct((e, H), F32),
            jax.ShapeDtypeStruct((e, H), F32),
        ),
        mesh=mesh,
        scratch_types=[
            pltpu.VMEM((CH,), jnp.int32),
            pltpu.VMEM((CH,), jnp.int32),
            pltpu.VMEM((CH, H), F32),
            pltpu.VMEM((CH, H), F32),
            pltpu.VMEM((CH, H), F32),
            pltpu.VMEM((CH, H), F32),
        ],
    )
    def k(p_hbm, q_hbm, x_hbm, row_hbm, col_hbm,
          pg_out, qg_out, xr_out, xc_out,
          ridx, cidx, pbuf, qbuf, xrbuf, xcbuf):
        c = lax.axis_index("c")
        s = lax.axis_index("s")
        wid = s * NC + c
        base = wid * e_per_w

        def body(j, carry):
            off = pl.multiple_of(base + j * CH, 8)
            pltpu.sync_copy(row_hbm.at[pl.ds(off, CH)], ridx)
            pltpu.sync_copy(col_hbm.at[pl.ds(off, CH)], cidx)
            pltpu.sync_copy(p_hbm.at[ridx], pbuf)
            pltpu.sync_copy(q_hbm.at[cidx], qbuf)
            pltpu.sync_copy(x_hbm.at[ridx], xrbuf)
            pltpu.sync_copy(x_hbm.at[cidx], xcbuf)
            pltpu.sync_copy(pbuf, pg_out.at[pl.ds(off, CH)])
            pltpu.sync_copy(qbuf, qg_out.at[pl.ds(off, CH)])
            pltpu.sync_copy(xrbuf, xr_out.at[pl.ds(off, CH)])
            pltpu.sync_copy(xcbuf, xc_out.at[pl.ds(off, CH)])
            return carry

        lax.fori_loop(0, n_sub, body, 0)

    return k(P, Q, X, row, col)


def _edge_call(pg, qg, xr, xc, ea, m_fold, w1c, beff, w2, b2, w3, b3, w4p, blk):
    e, H = pg.shape
    ein = ea.shape[1]

    def body(pg_ref, qg_ref, xr_ref, xc_ref, ea_ref, mf_ref, w1c_ref,
             beff_ref, w2_ref, b2_ref, w3_ref, b3_ref, w4_ref,
             m_ref, tr_ref):
        d = xr_ref[...] - xc_ref[...]
        dsq = jnp.sum(d * d, axis=1, keepdims=True)
        pre = (pg_ref[...] + qg_ref[...]
               + _dot(ea_ref[...], mf_ref[...])
               + dsq * w1c_ref[...] + beff_ref[...])
        u = _silu(pre)
        m = _silu(_dot(u, w2_ref[...]) + b2_ref[...])
        cw = _silu(_dot(m, w3_ref[...]) + b3_ref[...])
        ws = jnp.tanh(_dot(cw, w4_ref[...]))[:, 0:1]
        dist = jnp.sqrt(dsq + 1e-8)
        tr_ref[...] = d * (ws / (dist + 1e-8))
        m_ref[...] = m

    return pl.pallas_call(
        body,
        grid=(e // blk,),
        in_specs=[
            pl.BlockSpec((blk, H), lambda i: (i, 0)),
            pl.BlockSpec((blk, H), lambda i: (i, 0)),
            pl.BlockSpec((blk, H), lambda i: (i, 0)),
            pl.BlockSpec((blk, H), lambda i: (i, 0)),
            pl.BlockSpec((blk, ein), lambda i: (i, 0)),
            pl.BlockSpec((ein, H), lambda i: (0, 0)),
            pl.BlockSpec((1, H), lambda i: (0, 0)),
            pl.BlockSpec((1, H), lambda i: (0, 0)),
            pl.BlockSpec((H, H), lambda i: (0, 0)),
            pl.BlockSpec((1, H), lambda i: (0, 0)),
            pl.BlockSpec((H, H), lambda i: (0, 0)),
            pl.BlockSpec((1, H), lambda i: (0, 0)),
            pl.BlockSpec((H, 8), lambda i: (0, 0)),
        ],
        out_specs=[
            pl.BlockSpec((blk, H), lambda i: (i, 0)),
            pl.BlockSpec((blk, H), lambda i: (i, 0)),
        ],
        out_shape=[
            jax.ShapeDtypeStruct((e, H), F32),
            jax.ShapeDtypeStruct((e, H), F32),
        ],
    )(pg, qg, xr, xc, ea, m_fold, w1c, beff, w2, b2, w3, b3, w4p)


def _sc_scatter_call(m, tr, row, zeros_m):
    e, H = m.shape
    n = zeros_m.shape[0]
    e_per_w = e // NW
    n_sub = e_per_w // CH
    mesh = plsc.VectorSubcoreMesh(core_axis_name="c", subcore_axis_name="s")

    @functools.partial(
        pl.kernel,
        out_type=(
            jax.ShapeDtypeStruct((NC * n, H), F32),
            jax.ShapeDtypeStruct((NC * n, H), F32),
        ),
        mesh=mesh,
        scratch_types=[
            pltpu.VMEM((CH,), jnp.int32),
            pltpu.VMEM((CH, H), F32),
            pltpu.VMEM_SHARED((n, H), F32),
        ],
    )
    def k(m_hbm, tr_hbm, row_hbm, zm_hbm,
          am_out, ax_out, ridx, mbuf, accm):
        c = lax.axis_index("c")
        s = lax.axis_index("s")
        wid = s * NC + c
        s0 = (n // NS) & ~7
        tail = n - NS * s0
        r0 = pl.multiple_of(s * s0, 8)

        def zero_acc():
            pltpu.sync_copy(zm_hbm.at[pl.ds(r0, s0)], accm.at[pl.ds(r0, s0)])
            if tail:
                @pl.when(s == NS - 1)
                def _():
                    pltpu.sync_copy(zm_hbm.at[pl.ds(NS * s0, tail)],
                                    accm.at[pl.ds(NS * s0, tail)])

        def scatter_phase(src_hbm):
            def body(j, carry):
                off = pl.multiple_of(wid * e_per_w + j * CH, 8)
                pltpu.sync_copy(row_hbm.at[pl.ds(off, CH)], ridx)
                pltpu.sync_copy(src_hbm.at[pl.ds(off, CH)], mbuf)
                pltpu.sync_copy(mbuf, accm.at[ridx], add=True)
                return carry
            lax.fori_loop(0, n_sub, body, 0)

        def dump(out_hbm):
            out_r0 = pl.multiple_of(c * n + r0, 8)
            pltpu.sync_copy(accm.at[pl.ds(r0, s0)],
                            out_hbm.at[pl.ds(out_r0, s0)])
            if tail:
                @pl.when(s == NS - 1)
                def _():
                    t0 = pl.multiple_of(c * n + NS * s0, 8)
                    pltpu.sync_copy(accm.at[pl.ds(NS * s0, tail)],
                                    out_hbm.at[pl.ds(t0, tail)])

        zero_acc()
        plsc.subcore_barrier()
        scatter_phase(m_hbm)
        plsc.subcore_barrier()
        dump(am_out)
        zero_acc()
        plsc.subcore_barrier()
        scatter_phase(tr_hbm)
        plsc.subcore_barrier()
        dump(ax_out)

    return k(m, tr, row, zeros_m)


def _node_call(hf, am, x, ax, mask128, w5h, w5a, b5, w6, b6, blk,
               x_in=None):
    n, H = hf.shape
    final = x_in is not None
    nb = n // blk

    def body(*refs):
        if final:
            (hf_ref, am0_ref, am1_ref, x_ref, ax0_ref, ax1_ref, mk_ref,
             w5h_ref, w5a_ref, b5_ref, w6_ref, b6_ref, xin_ref,
             ho_ref, xo_ref, v_ref) = refs
        else:
            (hf_ref, am0_ref, am1_ref, x_ref, ax0_ref, ax1_ref, mk_ref,
             w5h_ref, w5a_ref, b5_ref, w6_ref, b6_ref,
             ho_ref, xo_ref) = refs
        hv = hf_ref[...]
        agg = am0_ref[...] + am1_ref[...]
        hn = _silu(_dot(hv, w5h_ref[...]) + _dot(agg, w5a_ref[...])
                   + b5_ref[...])
        hn = _dot(hn, w6_ref[...]) + b6_ref[...]
        ho_ref[...] = hv + hn
        mk = mk_ref[...]
        xnew = x_ref[...] + (ax0_ref[...] + ax1_ref[...]) * mk
        xo_ref[...] = xnew
        if final:
            v_ref[...] = (xnew - xin_ref[...]) * mk

    in_specs = [
        pl.BlockSpec((blk, H), lambda i: (i, 0)),
        pl.BlockSpec((blk, H), lambda i: (i, 0)),
        pl.BlockSpec((blk, H), lambda i: (i + nb, 0)),
        pl.BlockSpec((blk, H), lambda i: (i, 0)),
        pl.BlockSpec((blk, H), lambda i: (i, 0)),
        pl.BlockSpec((blk, H), lambda i: (i + nb, 0)),
        pl.BlockSpec((blk, H), lambda i: (i, 0)),
        pl.BlockSpec((H, H), lambda i: (0, 0)),
        pl.BlockSpec((H, H), lambda i: (0, 0)),
        pl.BlockSpec((1, H), lambda i: (0, 0)),
        pl.BlockSpec((H, H), lambda i: (0, 0)),
        pl.BlockSpec((1, H), lambda i: (0, 0)),
    ]
    out_specs = [
        pl.BlockSpec((blk, H), lambda i: (i, 0)),
        pl.BlockSpec((blk, H), lambda i: (i, 0)),
    ]
    out_shape = [
        jax.ShapeDtypeStruct((n, H), F32),
        jax.ShapeDtypeStruct((n, H), F32),
    ]
    args = [hf, am, am, x, ax, ax, mask128, w5h, w5a, b5, w6, b6]
    if final:
        in_specs.append(pl.BlockSpec((blk, H), lambda i: (i, 0)))
        out_specs.append(pl.BlockSpec((blk, H), lambda i: (i, 0)))
        out_shape.append(jax.ShapeDtypeStruct((n, H), F32))
        args.append(x_in)

    return pl.pallas_call(
        body,
        grid=(nb,),
        in_specs=in_specs,
        out_specs=out_specs,
        out_shape=out_shape,
    )(*args)


def kernel(h, x, edge_index, edge_attr, t, mask, params):
    n, node_in = h.shape
    H = params["node_embed"]["W"].shape[1]
    num_layers = len(params["layers"])
    blk = 1000
    eblk = 2000

    row = edge_index[0]
    col = edge_index[1]
    x128 = jnp.pad(x, ((0, 0), (0, H - x.shape[1])))
    mask2 = mask if mask.ndim == 2 else mask[:, None]
    mask128 = jnp.broadcast_to(mask2, (n, H)).astype(F32)

    tp = params["time_mlp"]
    wt1 = tp[0]["W"].reshape(1, H)
    bt1 = tp[0]["b"].reshape(1, H)
    wt2 = tp[1]["W"]
    bt2 = tp[1]["b"].reshape(1, H)
    we = params["edge_embed"]["W"]
    be = params["edge_embed"]["b"].reshape(1, H)
    wn = params["node_embed"]["W"]
    bn = params["node_embed"]["b"].reshape(1, H)

    w1_stack = jnp.stack([lp["edge_mlp"][0]["W"] for lp in params["layers"]])
    b1_stack = jnp.stack([lp["edge_mlp"][0]["b"].reshape(1, H)
                          for lp in params["layers"]])
    w1d_stack = w1_stack[:, 2 * H + 1:, :]

    te, m_fold_stack, beff_stack = _prep_call(
        t, wt1, bt1, wt2, bt2, we, w1d_stack, be, b1_stack, num_layers)

    hf = _embed_call(h, wn, bn, te, blk)

    zeros_m = jnp.zeros((n, H), F32)

    x_in = x128
    v = None
    for l, lp in enumerate(params["layers"]):
        w1 = lp["edge_mlp"][0]["W"]
        w1a = w1[:H, :]
        w1b = w1[H:2 * H, :]
        w1c = w1[2 * H:2 * H + 1, :]
        w2 = lp["edge_mlp"][1]["W"]
        b2 = lp["edge_mlp"][1]["b"].reshape(1, H)
        w3 = lp["coord_mlp"][0]["W"]
        b3 = lp["coord_mlp"][0]["b"].reshape(1, H)
        w4p = jnp.pad(lp["coord_mlp"][1]["W"], ((0, 0), (0, 7)))
        w5 = lp["node_mlp"][0]["W"]
        w5h = w5[:H, :]
        w5a = w5[H:, :]
        b5 = lp["node_mlp"][0]["b"].reshape(1, H)
        w6 = lp["node_mlp"][1]["W"]
        b6 = lp["node_mlp"][1]["b"].reshape(1, H)

        P, Q = _proj_call(hf, w1a, w1b, blk)
        pg, qg, xr, xc = _sc_gather_call(P, Q, x128, row, col)
        m, tr = _edge_call(pg, qg, xr, xc, edge_attr,
                           m_fold_stack[l], w1c, beff_stack[l],
                           w2, b2, w3, b3, w4p, eblk)
        am, ax = _sc_scatter_call(m, tr, row, zeros_m)
        last = l == num_layers - 1
        outs = _node_call(hf, am, x128, ax, mask128, w5h, w5a, b5, w6, b6,
                          blk, x_in=x_in if last else None)
        if last:
            hf, x128, v = outs
        else:
            hf, x128 = outs

    return v[:, :3]


# speedup vs baseline: 3.7378x; 1.7412x over previous
import functools

import jax
import jax.numpy as jnp
from jax import lax
from jax.experimental import pallas as pl
from jax.experimental.pallas import tpu as pltpu
from jax.experimental.pallas import tpu_sc as plsc

F32 = jnp.float32

NC = 2
NS = 16
NW = NC * NS

CH = 80


def _silu(x):
    return x * jax.nn.sigmoid(x)


def _dot(a, b):
    return jnp.dot(a, b, preferred_element_type=F32)


def _prep_call(t, wt1, bt1, wt2, bt2, we, w1d_stack, be, b1_stack, num_layers):
    def body(t_ref, wt1_ref, bt1_ref, wt2_ref, bt2_ref, we_ref, w1d_ref,
             be_ref, b1_ref, te_ref, m_ref, beff_ref):
        tval = t_ref[0]
        u = _silu(tval * wt1_ref[...] + bt1_ref[...])
        te_ref[...] = _dot(u, wt2_ref[...]) + bt2_ref[...]
        for l in range(num_layers):
            w1d = w1d_ref[l]
            m_ref[l] = _dot(we_ref[...], w1d)
            beff_ref[l] = _dot(be_ref[...], w1d) + b1_ref[l]

    H = wt2.shape[0]
    return pl.pallas_call(
        body,
        in_specs=[
            pl.BlockSpec(memory_space=pltpu.SMEM),
            pl.BlockSpec((1, H), lambda: (0, 0)),
            pl.BlockSpec((1, H), lambda: (0, 0)),
            pl.BlockSpec((H, H), lambda: (0, 0)),
            pl.BlockSpec((1, H), lambda: (0, 0)),
            pl.BlockSpec((16, H), lambda: (0, 0)),
            pl.BlockSpec((num_layers, H, H), lambda: (0, 0, 0)),
            pl.BlockSpec((1, H), lambda: (0, 0)),
            pl.BlockSpec((num_layers, 1, H), lambda: (0, 0, 0)),
        ],
        out_specs=[
            pl.BlockSpec((1, H), lambda: (0, 0)),
            pl.BlockSpec((num_layers, 16, H), lambda: (0, 0, 0)),
            pl.BlockSpec((num_layers, 1, H), lambda: (0, 0, 0)),
        ],
        out_shape=[
            jax.ShapeDtypeStruct((1, H), F32),
            jax.ShapeDtypeStruct((num_layers, 16, H), F32),
            jax.ShapeDtypeStruct((num_layers, 1, H), F32),
        ],
    )(t, wt1, bt1, wt2, bt2, we, w1d_stack, be, b1_stack)


def _embed_call(h, wn, bn, te, blk):
    n, din = h.shape
    H = wn.shape[1]

    def body(h_ref, wn_ref, bn_ref, te_ref, out_ref):
        out_ref[...] = (_dot(h_ref[...], wn_ref[...]) + bn_ref[...]
                        + te_ref[...])

    return pl.pallas_call(
        body,
        grid=(n // blk,),
        in_specs=[
            pl.BlockSpec((blk, din), lambda i: (i, 0)),
            pl.BlockSpec((din, H), lambda i: (0, 0)),
            pl.BlockSpec((1, H), lambda i: (0, 0)),
            pl.BlockSpec((1, H), lambda i: (0, 0)),
        ],
        out_specs=pl.BlockSpec((blk, H), lambda i: (i, 0)),
        out_shape=jax.ShapeDtypeStruct((n, H), F32),
    )(h, wn, bn, te)


def _proj_call(hf, w1a, w1b, blk):
    n, H = hf.shape

    def body(h_ref, wa_ref, wb_ref, p_ref, q_ref):
        hv = h_ref[...]
        p_ref[...] = _dot(hv, wa_ref[...])
        q_ref[...] = _dot(hv, wb_ref[...])

    return pl.pallas_call(
        body,
        grid=(n // blk,),
        in_specs=[
            pl.BlockSpec((blk, H), lambda i: (i, 0)),
            pl.BlockSpec((H, H), lambda i: (0, 0)),
            pl.BlockSpec((H, H), lambda i: (0, 0)),
        ],
        out_specs=[
            pl.BlockSpec((blk, H), lambda i: (i, 0)),
            pl.BlockSpec((blk, H), lambda i: (i, 0)),
        ],
        out_shape=[
            jax.ShapeDtypeStruct((n, H), F32),
            jax.ShapeDtypeStruct((n, H), F32),
        ],
    )(hf, w1a, w1b)


def _sc_gather_call(P, Q, X, row, col):
    n, H = P.shape
    e = row.shape[0]
    e_per_w = e // NW
    n_sub = e_per_w // CH
    mesh = plsc.VectorSubcoreMesh(core_axis_name="c", subcore_axis_name="s")

    D = 3

    @functools.partial(
        pl.kernel,
        out_type=(
            jax.ShapeDtypeStruct((e, H), F32),
            jax.ShapeDtypeStruct((e, H), F32),
            jax.ShapeDtypeStruct((e, H), F32),
            jax.ShapeDtypeStruct((e, H), F32),
        ),
        mesh=mesh,
        scratch_types=[
            pltpu.VMEM((D, CH), jnp.int32),
            pltpu.VMEM((D, CH), jnp.int32),
            pltpu.VMEM((D, CH, H), F32),
            pltpu.VMEM((D, CH, H), F32),
            pltpu.VMEM((D, CH, H), F32),
            pltpu.VMEM((D, CH, H), F32),
        ] + [pltpu.SemaphoreType.DMA] * (3 * D),
    )
    def k(p_hbm, q_hbm, x_hbm, row_hbm, col_hbm,
          pg_out, qg_out, xr_out, xc_out,
          ridx, cidx, pbuf, qbuf, xrbuf, xcbuf, *sems):
        c = lax.axis_index("c")
        s = lax.axis_index("s")
        wid = s * NC + c
        base = wid * e_per_w
        isem = sems[0:D]
        gsem = sems[D:2 * D]
        wsem = sems[2 * D:3 * D]
        bufs = (pbuf, qbuf, xrbuf, xcbuf)
        outs = (pg_out, qg_out, xr_out, xc_out)

        def off_at(jj):
            return pl.multiple_of(base + jj * CH, 8)

        def start_idx(b, jj):
            off = off_at(jj)
            pltpu.async_copy(row_hbm.at[pl.ds(off, CH)], ridx.at[b], isem[b])
            pltpu.async_copy(col_hbm.at[pl.ds(off, CH)], cidx.at[b], isem[b])

        def wait_idx(b):
            for _ in range(2):
                pltpu.make_async_copy(row_hbm.at[pl.ds(0, CH)],
                                      ridx.at[b], isem[b]).wait()

        def start_gathers(b):
            pltpu.async_copy(p_hbm.at[ridx.at[b]], pbuf.at[b], gsem[b])
            pltpu.async_copy(q_hbm.at[cidx.at[b]], qbuf.at[b], gsem[b])
            pltpu.async_copy(x_hbm.at[ridx.at[b]], xrbuf.at[b], gsem[b])
            pltpu.async_copy(x_hbm.at[cidx.at[b]], xcbuf.at[b], gsem[b])

        def wait_gathers(b):
            for _ in range(4):
                pltpu.make_async_copy(p_hbm.at[pl.ds(0, CH)],
                                      pbuf.at[b], gsem[b]).wait()

        def start_writes(b, jj):
            off = off_at(jj)
            for buf, out in zip(bufs, outs):
                pltpu.async_copy(buf.at[b], out.at[pl.ds(off, CH)], wsem[b])

        def wait_writes(b):
            for _ in range(4):
                pltpu.make_async_copy(pbuf.at[b], pg_out.at[pl.ds(0, CH)],
                                      wsem[b]).wait()

        for b in range(D):
            start_idx(b, b)
        for b in range(D):
            wait_idx(b)
            start_gathers(b)

        nk = n_sub // D

        def body(kD, carry):
            for b in range(D):
                jj = D * kD + b
                wait_gathers(b)
                start_writes(b, jj)
                nxt = jj + D

                @pl.when(nxt <= n_sub - 1)
                def _():
                    start_idx(b, nxt)
                    wait_idx(b)
                    wait_writes(b)
                    start_gathers(b)
            return carry

        lax.fori_loop(0, nk, body, 0)
        for jj in range(D * nk, n_sub):
            b = jj % D
            wait_gathers(b)
            start_writes(b, jj)
        for b in range(D):
            wait_writes(b)

    return k(P, Q, X, row, col)


def _edge_call(pg, qg, xr, xc, ea, m_fold, w1c, beff, w2, b2, w3, b3, w4p, blk):
    e, H = pg.shape
    ein = ea.shape[1]

    def body(pg_ref, qg_ref, xr_ref, xc_ref, ea_ref, mf_ref, w1c_ref,
             beff_ref, w2_ref, b2_ref, w3_ref, b3_ref, w4_ref,
             m_ref, tr_ref):
        d = xr_ref[...] - xc_ref[...]
        dsq = jnp.sum(d * d, axis=1, keepdims=True)
        pre = (pg_ref[...] + qg_ref[...]
               + _dot(ea_ref[...], mf_ref[...])
               + dsq * w1c_ref[...] + beff_ref[...])
        u = _silu(pre)
        m = _silu(_dot(u, w2_ref[...]) + b2_ref[...])
        cw = _silu(_dot(m, w3_ref[...]) + b3_ref[...])
        ws = jnp.tanh(_dot(cw, w4_ref[...]))[:, 0:1]
        dist = jnp.sqrt(dsq + 1e-8)
        tr_ref[...] = d * (ws / (dist + 1e-8))
        m_ref[...] = m

    return pl.pallas_call(
        body,
        grid=(e // blk,),
        in_specs=[
            pl.BlockSpec((blk, H), lambda i: (i, 0)),
            pl.BlockSpec((blk, H), lambda i: (i, 0)),
            pl.BlockSpec((blk, H), lambda i: (i, 0)),
            pl.BlockSpec((blk, H), lambda i: (i, 0)),
            pl.BlockSpec((blk, ein), lambda i: (i, 0)),
            pl.BlockSpec((ein, H), lambda i: (0, 0)),
            pl.BlockSpec((1, H), lambda i: (0, 0)),
            pl.BlockSpec((1, H), lambda i: (0, 0)),
            pl.BlockSpec((H, H), lambda i: (0, 0)),
            pl.BlockSpec((1, H), lambda i: (0, 0)),
            pl.BlockSpec((H, H), lambda i: (0, 0)),
            pl.BlockSpec((1, H), lambda i: (0, 0)),
            pl.BlockSpec((H, 8), lambda i: (0, 0)),
        ],
        out_specs=[
            pl.BlockSpec((blk, H), lambda i: (i, 0)),
            pl.BlockSpec((blk, H), lambda i: (i, 0)),
        ],
        out_shape=[
            jax.ShapeDtypeStruct((e, H), F32),
            jax.ShapeDtypeStruct((e, H), F32),
        ],
    )(pg, qg, xr, xc, ea, m_fold, w1c, beff, w2, b2, w3, b3, w4p)


def _sc_scatter_call(m, tr, row, zeros_m):
    e, H = m.shape
    n = zeros_m.shape[0]
    e_per_w = e // NW
    n_sub = e_per_w // CH
    mesh = plsc.VectorSubcoreMesh(core_axis_name="c", subcore_axis_name="s")

    @functools.partial(
        pl.kernel,
        out_type=(
            jax.ShapeDtypeStruct((NC * n, H), F32),
            jax.ShapeDtypeStruct((NC * n, H), F32),
        ),
        mesh=mesh,
        scratch_types=[
            pltpu.VMEM((4, CH), jnp.int32),
            pltpu.VMEM((4, CH, H), F32),
            pltpu.VMEM_SHARED((n, H), F32),
        ] + [pltpu.SemaphoreType.DMA] * 8,
    )
    def k(m_hbm, tr_hbm, row_hbm, zm_hbm,
          am_out, ax_out, ridx, mbuf, accm, *sems):
        D = 4
        c = lax.axis_index("c")
        s = lax.axis_index("s")
        wid = s * NC + c
        s0 = (n // NS) & ~7
        tail = n - NS * s0
        r0 = pl.multiple_of(s * s0, 8)
        rsem = sems[0:D]
        asem = sems[D:2 * D]

        def zero_acc():
            pltpu.sync_copy(zm_hbm.at[pl.ds(r0, s0)], accm.at[pl.ds(r0, s0)])
            if tail:
                @pl.when(s == NS - 1)
                def _():
                    pltpu.sync_copy(zm_hbm.at[pl.ds(NS * s0, tail)],
                                    accm.at[pl.ds(NS * s0, tail)])

        def scatter_phase(src_hbm):
            def start_reads(b, jj):
                off = pl.multiple_of(wid * e_per_w + jj * CH, 8)
                pltpu.async_copy(row_hbm.at[pl.ds(off, CH)],
                                 ridx.at[b], rsem[b])
                pltpu.async_copy(src_hbm.at[pl.ds(off, CH)],
                                 mbuf.at[b], rsem[b])

            def wait_reads(b):
                pltpu.make_async_copy(row_hbm.at[pl.ds(0, CH)],
                                      ridx.at[b], rsem[b]).wait()
                pltpu.make_async_copy(src_hbm.at[pl.ds(0, CH)],
                                      mbuf.at[b], rsem[b]).wait()

            def start_add(b):
                pltpu.async_copy(mbuf.at[b], accm.at[ridx.at[b]],
                                 asem[b], add=True)

            def wait_add(b):
                pltpu.make_async_copy(mbuf.at[b], accm.at[pl.ds(0, CH)],
                                      asem[b]).wait()

            for b in range(D):
                start_reads(b, b)
            nk = n_sub // D

            def body(kD, carry):
                for b in range(D):
                    jj = D * kD + b
                    wait_reads(b)
                    start_add(b)
                    nxt = jj + D

                    @pl.when(nxt <= n_sub - 1)
                    def _():
                        wait_add(b)
                        start_reads(b, nxt)
                return carry

            lax.fori_loop(0, nk, body, 0)
            for jj in range(D * nk, n_sub):
                b = jj % D
                wait_reads(b)
                start_add(b)
            for b in range(D):
                wait_add(b)

        def dump(out_hbm):
            out_r0 = pl.multiple_of(c * n + r0, 8)
            pltpu.sync_copy(accm.at[pl.ds(r0, s0)],
                            out_hbm.at[pl.ds(out_r0, s0)])
            if tail:
                @pl.when(s == NS - 1)
                def _():
                    t0 = pl.multiple_of(c * n + NS * s0, 8)
                    pltpu.sync_copy(accm.at[pl.ds(NS * s0, tail)],
                                    out_hbm.at[pl.ds(t0, tail)])

        zero_acc()
        plsc.subcore_barrier()
        scatter_phase(m_hbm)
        plsc.subcore_barrier()
        dump(am_out)
        zero_acc()
        plsc.subcore_barrier()
        scatter_phase(tr_hbm)
        plsc.subcore_barrier()
        dump(ax_out)

    return k(m, tr, row, zeros_m)


def _node_call(hf, am, x, ax, mask128, w5h, w5a, b5, w6, b6, blk,
               x_in=None):
    n, H = hf.shape
    final = x_in is not None
    nb = n // blk

    def body(*refs):
        if final:
            (hf_ref, am0_ref, am1_ref, x_ref, ax0_ref, ax1_ref, mk_ref,
             w5h_ref, w5a_ref, b5_ref, w6_ref, b6_ref, xin_ref,
             ho_ref, xo_ref, v_ref) = refs
        else:
            (hf_ref, am0_ref, am1_ref, x_ref, ax0_ref, ax1_ref, mk_ref,
             w5h_ref, w5a_ref, b5_ref, w6_ref, b6_ref,
             ho_ref, xo_ref) = refs
        hv = hf_ref[...]
        agg = am0_ref[...] + am1_ref[...]
        hn = _silu(_dot(hv, w5h_ref[...]) + _dot(agg, w5a_ref[...])
                   + b5_ref[...])
        hn = _dot(hn, w6_ref[...]) + b6_ref[...]
        ho_ref[...] = hv + hn
        mk = mk_ref[...]
        xnew = x_ref[...] + (ax0_ref[...] + ax1_ref[...]) * mk
        xo_ref[...] = xnew
        if final:
            v_ref[...] = (xnew - xin_ref[...]) * mk

    in_specs = [
        pl.BlockSpec((blk, H), lambda i: (i, 0)),
        pl.BlockSpec((blk, H), lambda i: (i, 0)),
        pl.BlockSpec((blk, H), lambda i: (i + nb, 0)),
        pl.BlockSpec((blk, H), lambda i: (i, 0)),
        pl.BlockSpec((blk, H), lambda i: (i, 0)),
        pl.BlockSpec((blk, H), lambda i: (i + nb, 0)),
        pl.BlockSpec((blk, H), lambda i: (i, 0)),
        pl.BlockSpec((H, H), lambda i: (0, 0)),
        pl.BlockSpec((H, H), lambda i: (0, 0)),
        pl.BlockSpec((1, H), lambda i: (0, 0)),
        pl.BlockSpec((H, H), lambda i: (0, 0)),
        pl.BlockSpec((1, H), lambda i: (0, 0)),
    ]
    out_specs = [
        pl.BlockSpec((blk, H), lambda i: (i, 0)),
        pl.BlockSpec((blk, H), lambda i: (i, 0)),
    ]
    out_shape = [
        jax.ShapeDtypeStruct((n, H), F32),
        jax.ShapeDtypeStruct((n, H), F32),
    ]
    args = [hf, am, am, x, ax, ax, mask128, w5h, w5a, b5, w6, b6]
    if final:
        in_specs.append(pl.BlockSpec((blk, H), lambda i: (i, 0)))
        out_specs.append(pl.BlockSpec((blk, H), lambda i: (i, 0)))
        out_shape.append(jax.ShapeDtypeStruct((n, H), F32))
        args.append(x_in)

    return pl.pallas_call(
        body,
        grid=(nb,),
        in_specs=in_specs,
        out_specs=out_specs,
        out_shape=out_shape,
    )(*args)


def kernel(h, x, edge_index, edge_attr, t, mask, params):
    n, node_in = h.shape
    H = params["node_embed"]["W"].shape[1]
    num_layers = len(params["layers"])
    blk = 1000
    eblk = 2000

    row = edge_index[0]
    col = edge_index[1]
    x128 = jnp.pad(x, ((0, 0), (0, H - x.shape[1])))
    mask2 = mask if mask.ndim == 2 else mask[:, None]
    mask128 = jnp.broadcast_to(mask2, (n, H)).astype(F32)

    tp = params["time_mlp"]
    wt1 = tp[0]["W"].reshape(1, H)
    bt1 = tp[0]["b"].reshape(1, H)
    wt2 = tp[1]["W"]
    bt2 = tp[1]["b"].reshape(1, H)
    we = params["edge_embed"]["W"]
    be = params["edge_embed"]["b"].reshape(1, H)
    wn = params["node_embed"]["W"]
    bn = params["node_embed"]["b"].reshape(1, H)

    w1_stack = jnp.stack([lp["edge_mlp"][0]["W"] for lp in params["layers"]])
    b1_stack = jnp.stack([lp["edge_mlp"][0]["b"].reshape(1, H)
                          for lp in params["layers"]])
    w1d_stack = w1_stack[:, 2 * H + 1:, :]

    te, m_fold_stack, beff_stack = _prep_call(
        t, wt1, bt1, wt2, bt2, we, w1d_stack, be, b1_stack, num_layers)

    hf = _embed_call(h, wn, bn, te, blk)

    zeros_m = jnp.zeros((n, H), F32)

    x_in = x128
    v = None
    for l, lp in enumerate(params["layers"]):
        w1 = lp["edge_mlp"][0]["W"]
        w1a = w1[:H, :]
        w1b = w1[H:2 * H, :]
        w1c = w1[2 * H:2 * H + 1, :]
        w2 = lp["edge_mlp"][1]["W"]
        b2 = lp["edge_mlp"][1]["b"].reshape(1, H)
        w3 = lp["coord_mlp"][0]["W"]
        b3 = lp["coord_mlp"][0]["b"].reshape(1, H)
        w4p = jnp.pad(lp["coord_mlp"][1]["W"], ((0, 0), (0, 7)))
        w5 = lp["node_mlp"][0]["W"]
        w5h = w5[:H, :]
        w5a = w5[H:, :]
        b5 = lp["node_mlp"][0]["b"].reshape(1, H)
        w6 = lp["node_mlp"][1]["W"]
        b6 = lp["node_mlp"][1]["b"].reshape(1, H)

        P, Q = _proj_call(hf, w1a, w1b, blk)
        pg, qg, xr, xc = _sc_gather_call(P, Q, x128, row, col)
        m, tr = _edge_call(pg, qg, xr, xc, edge_attr,
                           m_fold_stack[l], w1c, beff_stack[l],
                           w2, b2, w3, b3, w4p, eblk)
        am, ax = _sc_scatter_call(m, tr, row, zeros_m)
        last = l == num_layers - 1
        outs = _node_call(hf, am, x128, ax, mask128, w5h, w5a, b5, w6, b6,
                          blk, x_in=x_in if last else None)
        if last:
            hf, x128, v = outs
        else:
            hf, x128 = outs

    return v[:, :3]


# speedup vs baseline: 4.0710x; 1.0892x over previous
import functools

import jax
import jax.numpy as jnp
from jax import lax
from jax.experimental import pallas as pl
from jax.experimental.pallas import tpu as pltpu
from jax.experimental.pallas import tpu_sc as plsc

F32 = jnp.float32

NC = 2
NS = 16
NW = NC * NS

CH = 80


def _silu(x):
    return x * jax.nn.sigmoid(x)


def _dot(a, b):
    return jnp.dot(a, b, preferred_element_type=F32)


def _prep_call(t, wt1, bt1, wt2, bt2, we, w1d_stack, be, b1_stack, num_layers):
    def body(t_ref, wt1_ref, bt1_ref, wt2_ref, bt2_ref, we_ref, w1d_ref,
             be_ref, b1_ref, te_ref, m_ref, beff_ref):
        tval = t_ref[0]
        u = _silu(tval * wt1_ref[...] + bt1_ref[...])
        te_ref[...] = _dot(u, wt2_ref[...]) + bt2_ref[...]
        for l in range(num_layers):
            w1d = w1d_ref[l]
            m_ref[l] = _dot(we_ref[...], w1d)
            beff_ref[l] = _dot(be_ref[...], w1d) + b1_ref[l]

    H = wt2.shape[0]
    return pl.pallas_call(
        body,
        in_specs=[
            pl.BlockSpec(memory_space=pltpu.SMEM),
            pl.BlockSpec((1, H), lambda: (0, 0)),
            pl.BlockSpec((1, H), lambda: (0, 0)),
            pl.BlockSpec((H, H), lambda: (0, 0)),
            pl.BlockSpec((1, H), lambda: (0, 0)),
            pl.BlockSpec((16, H), lambda: (0, 0)),
            pl.BlockSpec((num_layers, H, H), lambda: (0, 0, 0)),
            pl.BlockSpec((1, H), lambda: (0, 0)),
            pl.BlockSpec((num_layers, 1, H), lambda: (0, 0, 0)),
        ],
        out_specs=[
            pl.BlockSpec((1, H), lambda: (0, 0)),
            pl.BlockSpec((num_layers, 16, H), lambda: (0, 0, 0)),
            pl.BlockSpec((num_layers, 1, H), lambda: (0, 0, 0)),
        ],
        out_shape=[
            jax.ShapeDtypeStruct((1, H), F32),
            jax.ShapeDtypeStruct((num_layers, 16, H), F32),
            jax.ShapeDtypeStruct((num_layers, 1, H), F32),
        ],
    )(t, wt1, bt1, wt2, bt2, we, w1d_stack, be, b1_stack)


def _embed_call(h, wn, bn, te, blk):
    n, din = h.shape
    H = wn.shape[1]

    def body(h_ref, wn_ref, bn_ref, te_ref, out_ref):
        out_ref[...] = (_dot(h_ref[...], wn_ref[...]) + bn_ref[...]
                        + te_ref[...])

    return pl.pallas_call(
        body,
        grid=(n // blk,),
        in_specs=[
            pl.BlockSpec((blk, din), lambda i: (i, 0)),
            pl.BlockSpec((din, H), lambda i: (0, 0)),
            pl.BlockSpec((1, H), lambda i: (0, 0)),
            pl.BlockSpec((1, H), lambda i: (0, 0)),
        ],
        out_specs=pl.BlockSpec((blk, H), lambda i: (i, 0)),
        out_shape=jax.ShapeDtypeStruct((n, H), F32),
    )(h, wn, bn, te)


def _proj_call(hf, w1a, w1b, blk):
    n, H = hf.shape

    def body(h_ref, wa_ref, wb_ref, p_ref, q_ref):
        hv = h_ref[...]
        p_ref[...] = _dot(hv, wa_ref[...])
        q_ref[...] = _dot(hv, wb_ref[...])

    return pl.pallas_call(
        body,
        grid=(n // blk,),
        in_specs=[
            pl.BlockSpec((blk, H), lambda i: (i, 0)),
            pl.BlockSpec((H, H), lambda i: (0, 0)),
            pl.BlockSpec((H, H), lambda i: (0, 0)),
        ],
        out_specs=[
            pl.BlockSpec((blk, H), lambda i: (i, 0)),
            pl.BlockSpec((blk, H), lambda i: (i, 0)),
        ],
        out_shape=[
            jax.ShapeDtypeStruct((n, H), F32),
            jax.ShapeDtypeStruct((n, H), F32),
        ],
    )(hf, w1a, w1b)


def _sc_gather_call(P, Q, X, row, col):
    n, H = P.shape
    e = row.shape[0]
    e_per_w = e // NW
    n_sub = e_per_w // CH
    mesh = plsc.VectorSubcoreMesh(core_axis_name="c", subcore_axis_name="s")

    D = 3

    @functools.partial(
        pl.kernel,
        out_type=(
            jax.ShapeDtypeStruct((e, H), F32),
            jax.ShapeDtypeStruct((e, H), F32),
            jax.ShapeDtypeStruct((e, H), F32),
        ),
        mesh=mesh,
        scratch_types=[
            pltpu.VMEM((D, CH), jnp.int32),
            pltpu.VMEM((D, CH), jnp.int32),
            pltpu.VMEM((D, CH, H), F32),
            pltpu.VMEM((D, CH, H), F32),
            pltpu.VMEM((D, CH, H), F32),
            pltpu.VMEM((D, CH, H), F32),
        ] + [pltpu.SemaphoreType.DMA] * (3 * D),
    )
    def k(p_hbm, q_hbm, x_hbm, row_hbm, col_hbm,
          pg_out, qg_out, df_out,
          ridx, cidx, pbuf, qbuf, xrbuf, xcbuf, *sems):
        c = lax.axis_index("c")
        s = lax.axis_index("s")
        wid = s * NC + c
        base = wid * e_per_w
        isem = sems[0:D]
        gsem = sems[D:2 * D]
        wsem = sems[2 * D:3 * D]
        bufs = (pbuf, qbuf, xrbuf)
        outs = (pg_out, qg_out, df_out)

        def off_at(jj):
            return pl.multiple_of(base + jj * CH, 8)

        def start_idx(b, jj):
            off = off_at(jj)
            pltpu.async_copy(row_hbm.at[pl.ds(off, CH)], ridx.at[b], isem[b])
            pltpu.async_copy(col_hbm.at[pl.ds(off, CH)], cidx.at[b], isem[b])

        def wait_idx(b):
            for _ in range(2):
                pltpu.make_async_copy(row_hbm.at[pl.ds(0, CH)],
                                      ridx.at[b], isem[b]).wait()

        def start_gathers(b):
            pltpu.async_copy(p_hbm.at[ridx.at[b]], pbuf.at[b], gsem[b])
            pltpu.async_copy(q_hbm.at[cidx.at[b]], qbuf.at[b], gsem[b])
            pltpu.async_copy(x_hbm.at[ridx.at[b]], xrbuf.at[b], gsem[b])
            pltpu.async_copy(x_hbm.at[cidx.at[b]], xcbuf.at[b], gsem[b])

        def wait_gathers(b):
            for _ in range(4):
                pltpu.make_async_copy(p_hbm.at[pl.ds(0, CH)],
                                      pbuf.at[b], gsem[b]).wait()

        def compute_diff(b):
            def sub_row(r, carry):
                xrbuf[b, r, pl.ds(0, 16)] = (xrbuf[b, r, pl.ds(0, 16)]
                                             - xcbuf[b, r, pl.ds(0, 16)])
                return carry
            lax.fori_loop(0, CH, sub_row, 0)

        def start_writes(b, jj):
            off = off_at(jj)
            for buf, out in zip(bufs, outs):
                pltpu.async_copy(buf.at[b], out.at[pl.ds(off, CH)], wsem[b])

        def wait_writes(b):
            for _ in range(3):
                pltpu.make_async_copy(pbuf.at[b], pg_out.at[pl.ds(0, CH)],
                                      wsem[b]).wait()

        for b in range(D):
            start_idx(b, b)
        for b in range(D):
            wait_idx(b)
            start_gathers(b)

        nk = n_sub // D

        def body(kD, carry):
            for b in range(D):
                jj = D * kD + b
                wait_gathers(b)
                compute_diff(b)
                start_writes(b, jj)
                nxt = jj + D

                @pl.when(nxt <= n_sub - 1)
                def _():
                    start_idx(b, nxt)
                    wait_idx(b)
                    wait_writes(b)
                    start_gathers(b)
            return carry

        lax.fori_loop(0, nk, body, 0)
        for jj in range(D * nk, n_sub):
            b = jj % D
            wait_gathers(b)
            compute_diff(b)
            start_writes(b, jj)
        for b in range(D):
            wait_writes(b)

    return k(P, Q, X, row, col)


def _edge_call(pg, qg, df, ea, m_fold, w1c, beff, w2, b2, w3, b3, w4p, blk):
    e, H = pg.shape
    ein = ea.shape[1]

    def body(pg_ref, qg_ref, df_ref, ea_ref, mf_ref, w1c_ref,
             beff_ref, w2_ref, b2_ref, w3_ref, b3_ref, w4_ref,
             m_ref, tr_ref):
        d = df_ref[...]
        dsq = jnp.sum(d * d, axis=1, keepdims=True)
        pre = (pg_ref[...] + qg_ref[...]
               + _dot(ea_ref[...], mf_ref[...])
               + dsq * w1c_ref[...] + beff_ref[...])
        u = _silu(pre)
        m = _silu(_dot(u, w2_ref[...]) + b2_ref[...])
        cw = _silu(_dot(m, w3_ref[...]) + b3_ref[...])
        ws = jnp.tanh(_dot(cw, w4_ref[...]))[:, 0:1]
        dist = jnp.sqrt(dsq + 1e-8)
        tr_ref[...] = d * (ws / (dist + 1e-8))
        m_ref[...] = m

    return pl.pallas_call(
        body,
        grid=(e // blk,),
        in_specs=[
            pl.BlockSpec((blk, H), lambda i: (i, 0)),
            pl.BlockSpec((blk, H), lambda i: (i, 0)),
            pl.BlockSpec((blk, H), lambda i: (i, 0)),
            pl.BlockSpec((blk, ein), lambda i: (i, 0)),
            pl.BlockSpec((ein, H), lambda i: (0, 0)),
            pl.BlockSpec((1, H), lambda i: (0, 0)),
            pl.BlockSpec((1, H), lambda i: (0, 0)),
            pl.BlockSpec((H, H), lambda i: (0, 0)),
            pl.BlockSpec((1, H), lambda i: (0, 0)),
            pl.BlockSpec((H, H), lambda i: (0, 0)),
            pl.BlockSpec((1, H), lambda i: (0, 0)),
            pl.BlockSpec((H, 8), lambda i: (0, 0)),
        ],
        out_specs=[
            pl.BlockSpec((blk, H), lambda i: (i, 0)),
            pl.BlockSpec((blk, H), lambda i: (i, 0)),
        ],
        out_shape=[
            jax.ShapeDtypeStruct((e, H), F32),
            jax.ShapeDtypeStruct((e, H), F32),
        ],
    )(pg, qg, df, ea, m_fold, w1c, beff, w2, b2, w3, b3, w4p)


def _sc_scatter_call(m, tr, row, zeros_m):
    e, H = m.shape
    n = zeros_m.shape[0]
    e_per_w = e // NW
    n_sub = e_per_w // CH
    mesh = plsc.VectorSubcoreMesh(core_axis_name="c", subcore_axis_name="s")

    @functools.partial(
        pl.kernel,
        out_type=(
            jax.ShapeDtypeStruct((NC * n, H), F32),
            jax.ShapeDtypeStruct((NC * n, H), F32),
        ),
        mesh=mesh,
        scratch_types=[
            pltpu.VMEM((4, CH), jnp.int32),
            pltpu.VMEM((4, CH, H), F32),
            pltpu.VMEM_SHARED((n, H), F32),
        ] + [pltpu.SemaphoreType.DMA] * 8,
    )
    def k(m_hbm, tr_hbm, row_hbm, zm_hbm,
          am_out, ax_out, ridx, mbuf, accm, *sems):
        D = 4
        c = lax.axis_index("c")
        s = lax.axis_index("s")
        wid = s * NC + c
        s0 = (n // NS) & ~7
        tail = n - NS * s0
        r0 = pl.multiple_of(s * s0, 8)
        rsem = sems[0:D]
        asem = sems[D:2 * D]

        def zero_acc():
            pltpu.sync_copy(zm_hbm.at[pl.ds(r0, s0)], accm.at[pl.ds(r0, s0)])
            if tail:
                @pl.when(s == NS - 1)
                def _():
                    pltpu.sync_copy(zm_hbm.at[pl.ds(NS * s0, tail)],
                                    accm.at[pl.ds(NS * s0, tail)])

        def scatter_phase(src_hbm):
            def start_reads(b, jj):
                off = pl.multiple_of(wid * e_per_w + jj * CH, 8)
                pltpu.async_copy(row_hbm.at[pl.ds(off, CH)],
                                 ridx.at[b], rsem[b])
                pltpu.async_copy(src_hbm.at[pl.ds(off, CH)],
                                 mbuf.at[b], rsem[b])

            def wait_reads(b):
                pltpu.make_async_copy(row_hbm.at[pl.ds(0, CH)],
                                      ridx.at[b], rsem[b]).wait()
                pltpu.make_async_copy(src_hbm.at[pl.ds(0, CH)],
                                      mbuf.at[b], rsem[b]).wait()

            def start_add(b):
                pltpu.async_copy(mbuf.at[b], accm.at[ridx.at[b]],
                                 asem[b], add=True)

            def wait_add(b):
                pltpu.make_async_copy(mbuf.at[b], accm.at[pl.ds(0, CH)],
                                      asem[b]).wait()

            for b in range(D):
                start_reads(b, b)
            nk = n_sub // D

            def body(kD, carry):
                for b in range(D):
                    jj = D * kD + b
                    wait_reads(b)
                    start_add(b)
                    nxt = jj + D

                    @pl.when(nxt <= n_sub - 1)
                    def _():
                        wait_add(b)
                        start_reads(b, nxt)
                return carry

            lax.fori_loop(0, nk, body, 0)
            for jj in range(D * nk, n_sub):
                b = jj % D
                wait_reads(b)
                start_add(b)
            for b in range(D):
                wait_add(b)

        def dump(out_hbm):
            out_r0 = pl.multiple_of(c * n + r0, 8)
            pltpu.sync_copy(accm.at[pl.ds(r0, s0)],
                            out_hbm.at[pl.ds(out_r0, s0)])
            if tail:
                @pl.when(s == NS - 1)
                def _():
                    t0 = pl.multiple_of(c * n + NS * s0, 8)
                    pltpu.sync_copy(accm.at[pl.ds(NS * s0, tail)],
                                    out_hbm.at[pl.ds(t0, tail)])

        zero_acc()
        plsc.subcore_barrier()
        scatter_phase(m_hbm)
        plsc.subcore_barrier()
        dump(am_out)
        zero_acc()
        plsc.subcore_barrier()
        scatter_phase(tr_hbm)
        plsc.subcore_barrier()
        dump(ax_out)

    return k(m, tr, row, zeros_m)


def _node_call(hf, am, x, ax, mask128, w5h, w5a, b5, w6, b6, blk,
               x_in=None):
    n, H = hf.shape
    final = x_in is not None
    nb = n // blk

    def body(*refs):
        if final:
            (hf_ref, am0_ref, am1_ref, x_ref, ax0_ref, ax1_ref, mk_ref,
             w5h_ref, w5a_ref, b5_ref, w6_ref, b6_ref, xin_ref,
             ho_ref, xo_ref, v_ref) = refs
        else:
            (hf_ref, am0_ref, am1_ref, x_ref, ax0_ref, ax1_ref, mk_ref,
             w5h_ref, w5a_ref, b5_ref, w6_ref, b6_ref,
             ho_ref, xo_ref) = refs
        hv = hf_ref[...]
        agg = am0_ref[...] + am1_ref[...]
        hn = _silu(_dot(hv, w5h_ref[...]) + _dot(agg, w5a_ref[...])
                   + b5_ref[...])
        hn = _dot(hn, w6_ref[...]) + b6_ref[...]
        ho_ref[...] = hv + hn
        mk = mk_ref[...]
        xnew = x_ref[...] + (ax0_ref[...] + ax1_ref[...]) * mk
        xo_ref[...] = xnew
        if final:
            v_ref[...] = (xnew - xin_ref[...]) * mk

    in_specs = [
        pl.BlockSpec((blk, H), lambda i: (i, 0)),
        pl.BlockSpec((blk, H), lambda i: (i, 0)),
        pl.BlockSpec((blk, H), lambda i: (i + nb, 0)),
        pl.BlockSpec((blk, H), lambda i: (i, 0)),
        pl.BlockSpec((blk, H), lambda i: (i, 0)),
        pl.BlockSpec((blk, H), lambda i: (i + nb, 0)),
        pl.BlockSpec((blk, H), lambda i: (i, 0)),
        pl.BlockSpec((H, H), lambda i: (0, 0)),
        pl.BlockSpec((H, H), lambda i: (0, 0)),
        pl.BlockSpec((1, H), lambda i: (0, 0)),
        pl.BlockSpec((H, H), lambda i: (0, 0)),
        pl.BlockSpec((1, H), lambda i: (0, 0)),
    ]
    out_specs = [
        pl.BlockSpec((blk, H), lambda i: (i, 0)),
        pl.BlockSpec((blk, H), lambda i: (i, 0)),
    ]
    out_shape = [
        jax.ShapeDtypeStruct((n, H), F32),
        jax.ShapeDtypeStruct((n, H), F32),
    ]
    args = [hf, am, am, x, ax, ax, mask128, w5h, w5a, b5, w6, b6]
    if final:
        in_specs.append(pl.BlockSpec((blk, H), lambda i: (i, 0)))
        out_specs.append(pl.BlockSpec((blk, H), lambda i: (i, 0)))
        out_shape.append(jax.ShapeDtypeStruct((n, H), F32))
        args.append(x_in)

    return pl.pallas_call(
        body,
        grid=(nb,),
        in_specs=in_specs,
        out_specs=out_specs,
        out_shape=out_shape,
    )(*args)


def kernel(h, x, edge_index, edge_attr, t, mask, params):
    n, node_in = h.shape
    H = params["node_embed"]["W"].shape[1]
    num_layers = len(params["layers"])
    blk = 1000
    eblk = 2000

    row = edge_index[0]
    col = edge_index[1]
    x128 = jnp.pad(x, ((0, 0), (0, H - x.shape[1])))
    mask2 = mask if mask.ndim == 2 else mask[:, None]
    mask128 = jnp.broadcast_to(mask2, (n, H)).astype(F32)

    tp = params["time_mlp"]
    wt1 = tp[0]["W"].reshape(1, H)
    bt1 = tp[0]["b"].reshape(1, H)
    wt2 = tp[1]["W"]
    bt2 = tp[1]["b"].reshape(1, H)
    we = params["edge_embed"]["W"]
    be = params["edge_embed"]["b"].reshape(1, H)
    wn = params["node_embed"]["W"]
    bn = params["node_embed"]["b"].reshape(1, H)

    w1_stack = jnp.stack([lp["edge_mlp"][0]["W"] for lp in params["layers"]])
    b1_stack = jnp.stack([lp["edge_mlp"][0]["b"].reshape(1, H)
                          for lp in params["layers"]])
    w1d_stack = w1_stack[:, 2 * H + 1:, :]

    te, m_fold_stack, beff_stack = _prep_call(
        t, wt1, bt1, wt2, bt2, we, w1d_stack, be, b1_stack, num_layers)

    hf = _embed_call(h, wn, bn, te, blk)

    zeros_m = jnp.zeros((n, H), F32)

    x_in = x128
    v = None
    for l, lp in enumerate(params["layers"]):
        w1 = lp["edge_mlp"][0]["W"]
        w1a = w1[:H, :]
        w1b = w1[H:2 * H, :]
        w1c = w1[2 * H:2 * H + 1, :]
        w2 = lp["edge_mlp"][1]["W"]
        b2 = lp["edge_mlp"][1]["b"].reshape(1, H)
        w3 = lp["coord_mlp"][0]["W"]
        b3 = lp["coord_mlp"][0]["b"].reshape(1, H)
        w4p = jnp.pad(lp["coord_mlp"][1]["W"], ((0, 0), (0, 7)))
        w5 = lp["node_mlp"][0]["W"]
        w5h = w5[:H, :]
        w5a = w5[H:, :]
        b5 = lp["node_mlp"][0]["b"].reshape(1, H)
        w6 = lp["node_mlp"][1]["W"]
        b6 = lp["node_mlp"][1]["b"].reshape(1, H)

        P, Q = _proj_call(hf, w1a, w1b, blk)
        pg, qg, df = _sc_gather_call(P, Q, x128, row, col)
        m, tr = _edge_call(pg, qg, df, edge_attr,
                           m_fold_stack[l], w1c, beff_stack[l],
                           w2, b2, w3, b3, w4p, eblk)
        am, ax = _sc_scatter_call(m, tr, row, zeros_m)
        last = l == num_layers - 1
        outs = _node_call(hf, am, x128, ax, mask128, w5h, w5a, b5, w6, b6,
                          blk, x_in=x_in if last else None)
        if last:
            hf, x128, v = outs
        else:
            hf, x128 = outs

    return v[:, :3]


# speedup vs baseline: 4.0829x; 1.0029x over previous
import functools

import jax
import jax.numpy as jnp
from jax import lax
from jax.experimental import pallas as pl
from jax.experimental.pallas import tpu as pltpu
from jax.experimental.pallas import tpu_sc as plsc

F32 = jnp.float32

NC = 2
NS = 16
NW = NC * NS

CH = 80


def _silu(x):
    return x * jax.nn.sigmoid(x)


def _dot(a, b):
    return jnp.dot(a, b, preferred_element_type=F32)


def _prep_call(t, wt1, bt1, wt2, bt2, we, w1d_stack, be, b1_stack, num_layers):
    def body(t_ref, wt1_ref, bt1_ref, wt2_ref, bt2_ref, we_ref, w1d_ref,
             be_ref, b1_ref, te_ref, m_ref, beff_ref):
        tval = t_ref[0]
        u = _silu(tval * wt1_ref[...] + bt1_ref[...])
        te_ref[...] = _dot(u, wt2_ref[...]) + bt2_ref[...]
        for l in range(num_layers):
            w1d = w1d_ref[l]
            m_ref[l] = _dot(we_ref[...], w1d)
            beff_ref[l] = _dot(be_ref[...], w1d) + b1_ref[l]

    H = wt2.shape[0]
    return pl.pallas_call(
        body,
        in_specs=[
            pl.BlockSpec(memory_space=pltpu.SMEM),
            pl.BlockSpec((1, H), lambda: (0, 0)),
            pl.BlockSpec((1, H), lambda: (0, 0)),
            pl.BlockSpec((H, H), lambda: (0, 0)),
            pl.BlockSpec((1, H), lambda: (0, 0)),
            pl.BlockSpec((16, H), lambda: (0, 0)),
            pl.BlockSpec((num_layers, H, H), lambda: (0, 0, 0)),
            pl.BlockSpec((1, H), lambda: (0, 0)),
            pl.BlockSpec((num_layers, 1, H), lambda: (0, 0, 0)),
        ],
        out_specs=[
            pl.BlockSpec((1, H), lambda: (0, 0)),
            pl.BlockSpec((num_layers, 16, H), lambda: (0, 0, 0)),
            pl.BlockSpec((num_layers, 1, H), lambda: (0, 0, 0)),
        ],
        out_shape=[
            jax.ShapeDtypeStruct((1, H), F32),
            jax.ShapeDtypeStruct((num_layers, 16, H), F32),
            jax.ShapeDtypeStruct((num_layers, 1, H), F32),
        ],
    )(t, wt1, bt1, wt2, bt2, we, w1d_stack, be, b1_stack)


def _embed_call(h, wn, bn, te, blk):
    n, din = h.shape
    H = wn.shape[1]

    def body(h_ref, wn_ref, bn_ref, te_ref, out_ref):
        out_ref[...] = (_dot(h_ref[...], wn_ref[...]) + bn_ref[...]
                        + te_ref[...])

    return pl.pallas_call(
        body,
        grid=(n // blk,),
        in_specs=[
            pl.BlockSpec((blk, din), lambda i: (i, 0)),
            pl.BlockSpec((din, H), lambda i: (0, 0)),
            pl.BlockSpec((1, H), lambda i: (0, 0)),
            pl.BlockSpec((1, H), lambda i: (0, 0)),
        ],
        out_specs=pl.BlockSpec((blk, H), lambda i: (i, 0)),
        out_shape=jax.ShapeDtypeStruct((n, H), F32),
    )(h, wn, bn, te)


def _proj_call(hf, w1a, w1b, blk):
    n, H = hf.shape

    def body(h_ref, wa_ref, wb_ref, p_ref, q_ref):
        hv = h_ref[...]
        p_ref[...] = _dot(hv, wa_ref[...])
        q_ref[...] = _dot(hv, wb_ref[...])

    return pl.pallas_call(
        body,
        grid=(n // blk,),
        in_specs=[
            pl.BlockSpec((blk, H), lambda i: (i, 0)),
            pl.BlockSpec((H, H), lambda i: (0, 0)),
            pl.BlockSpec((H, H), lambda i: (0, 0)),
        ],
        out_specs=[
            pl.BlockSpec((blk, H), lambda i: (i, 0)),
            pl.BlockSpec((blk, H), lambda i: (i, 0)),
        ],
        out_shape=[
            jax.ShapeDtypeStruct((n, H), F32),
            jax.ShapeDtypeStruct((n, H), F32),
        ],
    )(hf, w1a, w1b)


def _sc_gather_call(P, Q, X, row, col):
    n, H = P.shape
    e = row.shape[0]
    e_per_w = e // NW
    n_sub = e_per_w // CH
    mesh = plsc.VectorSubcoreMesh(core_axis_name="c", subcore_axis_name="s")

    D = 2

    @functools.partial(
        pl.kernel,
        out_type=(
            jax.ShapeDtypeStruct((e, H), F32),
            jax.ShapeDtypeStruct((e, H), F32),
            jax.ShapeDtypeStruct((e, H), F32),
        ),
        mesh=mesh,
        scratch_types=[
            pltpu.VMEM((D, CH), jnp.int32),
            pltpu.VMEM((D, CH), jnp.int32),
            pltpu.VMEM((D, CH, H), F32),
            pltpu.VMEM((D, CH, H), F32),
            pltpu.VMEM((D, CH, H), F32),
            pltpu.VMEM((D, CH, H), F32),
            pltpu.VMEM((D, CH, H), F32),
        ] + [pltpu.SemaphoreType.DMA] * (3 * D),
    )
    def k(p_hbm, q_hbm, x_hbm, row_hbm, col_hbm,
          pg_out, qg_out, df_out,
          ridx, cidx, pbuf, qbuf, xrbuf, xcbuf, dfbuf, *sems):
        c = lax.axis_index("c")
        s = lax.axis_index("s")
        wid = s * NC + c
        base = wid * e_per_w
        isem = sems[0:D]
        gsem = sems[D:2 * D]
        wsem = sems[2 * D:3 * D]
        bufs = (pbuf, qbuf, dfbuf)
        outs = (pg_out, qg_out, df_out)

        zeros16 = jnp.zeros((16,), F32)

        def zrow(i, carry):
            for b in range(D):
                for u in range(H // 16):
                    dfbuf[b, i, pl.ds(16 * u, 16)] = zeros16
            return carry

        lax.fori_loop(0, CH, zrow, 0)

        def off_at(jj):
            return pl.multiple_of(base + jj * CH, 8)

        def start_idx(b, jj):
            off = off_at(jj)
            pltpu.async_copy(row_hbm.at[pl.ds(off, CH)], ridx.at[b], isem[b])
            pltpu.async_copy(col_hbm.at[pl.ds(off, CH)], cidx.at[b], isem[b])

        def wait_idx(b):
            for _ in range(2):
                pltpu.make_async_copy(row_hbm.at[pl.ds(0, CH)],
                                      ridx.at[b], isem[b]).wait()

        def start_gathers(b):
            pltpu.async_copy(p_hbm.at[ridx.at[b]], pbuf.at[b], gsem[b])
            pltpu.async_copy(q_hbm.at[cidx.at[b]], qbuf.at[b], gsem[b])
            pltpu.async_copy(x_hbm.at[ridx.at[b]], xrbuf.at[b], gsem[b])
            pltpu.async_copy(x_hbm.at[cidx.at[b]], xcbuf.at[b], gsem[b])

        def wait_gathers(b):
            for _ in range(4):
                pltpu.make_async_copy(p_hbm.at[pl.ds(0, CH)],
                                      pbuf.at[b], gsem[b]).wait()

        def compute_diff(b):
            def sub_row(r, carry):
                dfbuf[b, r, pl.ds(0, 16)] = (xrbuf[b, r, pl.ds(0, 16)]
                                             - xcbuf[b, r, pl.ds(0, 16)])
                return carry
            lax.fori_loop(0, CH, sub_row, 0)

        def start_writes(b, jj):
            off = off_at(jj)
            for buf, out in zip(bufs, outs):
                pltpu.async_copy(buf.at[b], out.at[pl.ds(off, CH)], wsem[b])

        def wait_writes(b):
            for _ in range(3):
                pltpu.make_async_copy(pbuf.at[b], pg_out.at[pl.ds(0, CH)],
                                      wsem[b]).wait()

        for b in range(D):
            start_idx(b, b)
        for b in range(D):
            wait_idx(b)
            start_gathers(b)

        nk = n_sub // D

        def body(kD, carry):
            for b in range(D):
                jj = D * kD + b
                wait_gathers(b)
                compute_diff(b)
                start_writes(b, jj)
                nxt = jj + D

                @pl.when(nxt <= n_sub - 1)
                def _():
                    start_idx(b, nxt)
                    wait_idx(b)
                    wait_writes(b)
                    start_gathers(b)
            return carry

        lax.fori_loop(0, nk, body, 0)
        for jj in range(D * nk, n_sub):
            b = jj % D
            wait_gathers(b)
            compute_diff(b)
            start_writes(b, jj)
        for b in range(D):
            wait_writes(b)

    return k(P, Q, X, row, col)


def _edge_call(pg, qg, df, ea, m_fold, w1c, beff, w2, b2, w3, b3, w4p, blk):
    e, H = pg.shape
    ein = ea.shape[1]

    def body(pg_ref, qg_ref, df_ref, ea_ref, mf_ref, w1c_ref,
             beff_ref, w2_ref, b2_ref, w3_ref, b3_ref, w4_ref,
             m_ref, tr_ref):
        d = df_ref[...]
        dsq = jnp.sum(d * d, axis=1, keepdims=True)
        pre = (pg_ref[...] + qg_ref[...]
               + _dot(ea_ref[...], mf_ref[...])
               + dsq * w1c_ref[...] + beff_ref[...])
        u = _silu(pre)
        m = _silu(_dot(u, w2_ref[...]) + b2_ref[...])
        cw = _silu(_dot(m, w3_ref[...]) + b3_ref[...])
        ws = jnp.tanh(_dot(cw, w4_ref[...]))[:, 0:1]
        dist = jnp.sqrt(dsq + 1e-8)
        tr_ref[...] = d * (ws / (dist + 1e-8))
        m_ref[...] = m

    return pl.pallas_call(
        body,
        grid=(e // blk,),
        in_specs=[
            pl.BlockSpec((blk, H), lambda i: (i, 0)),
            pl.BlockSpec((blk, H), lambda i: (i, 0)),
            pl.BlockSpec((blk, H), lambda i: (i, 0)),
            pl.BlockSpec((blk, ein), lambda i: (i, 0)),
            pl.BlockSpec((ein, H), lambda i: (0, 0)),
            pl.BlockSpec((1, H), lambda i: (0, 0)),
            pl.BlockSpec((1, H), lambda i: (0, 0)),
            pl.BlockSpec((H, H), lambda i: (0, 0)),
            pl.BlockSpec((1, H), lambda i: (0, 0)),
            pl.BlockSpec((H, H), lambda i: (0, 0)),
            pl.BlockSpec((1, H), lambda i: (0, 0)),
            pl.BlockSpec((H, 8), lambda i: (0, 0)),
        ],
        out_specs=[
            pl.BlockSpec((blk, H), lambda i: (i, 0)),
            pl.BlockSpec((blk, H), lambda i: (i, 0)),
        ],
        out_shape=[
            jax.ShapeDtypeStruct((e, H), F32),
            jax.ShapeDtypeStruct((e, H), F32),
        ],
    )(pg, qg, df, ea, m_fold, w1c, beff, w2, b2, w3, b3, w4p)


def _sc_scatter_call(m, tr, row, zeros_m):
    e, H = m.shape
    n = zeros_m.shape[0]
    e_per_w = e // NW
    n_sub = e_per_w // CH
    mesh = plsc.VectorSubcoreMesh(core_axis_name="c", subcore_axis_name="s")

    @functools.partial(
        pl.kernel,
        out_type=(
            jax.ShapeDtypeStruct((NC * n, H), F32),
            jax.ShapeDtypeStruct((NC * n, H), F32),
        ),
        mesh=mesh,
        scratch_types=[
            pltpu.VMEM((4, CH), jnp.int32),
            pltpu.VMEM((4, CH, H), F32),
            pltpu.VMEM_SHARED((n, H), F32),
        ] + [pltpu.SemaphoreType.DMA] * 8,
    )
    def k(m_hbm, tr_hbm, row_hbm, zm_hbm,
          am_out, ax_out, ridx, mbuf, accm, *sems):
        D = 4
        c = lax.axis_index("c")
        s = lax.axis_index("s")
        wid = s * NC + c
        s0 = (n // NS) & ~7
        tail = n - NS * s0
        r0 = pl.multiple_of(s * s0, 8)
        rsem = sems[0:D]
        asem = sems[D:2 * D]

        def zero_acc():
            pltpu.sync_copy(zm_hbm.at[pl.ds(r0, s0)], accm.at[pl.ds(r0, s0)])
            if tail:
                @pl.when(s == NS - 1)
                def _():
                    pltpu.sync_copy(zm_hbm.at[pl.ds(NS * s0, tail)],
                                    accm.at[pl.ds(NS * s0, tail)])

        def scatter_phase(src_hbm):
            def start_reads(b, jj):
                off = pl.multiple_of(wid * e_per_w + jj * CH, 8)
                pltpu.async_copy(row_hbm.at[pl.ds(off, CH)],
                                 ridx.at[b], rsem[b])
                pltpu.async_copy(src_hbm.at[pl.ds(off, CH)],
                                 mbuf.at[b], rsem[b])

            def wait_reads(b):
                pltpu.make_async_copy(row_hbm.at[pl.ds(0, CH)],
                                      ridx.at[b], rsem[b]).wait()
                pltpu.make_async_copy(src_hbm.at[pl.ds(0, CH)],
                                      mbuf.at[b], rsem[b]).wait()

            def start_add(b):
                pltpu.async_copy(mbuf.at[b], accm.at[ridx.at[b]],
                                 asem[b], add=True)

            def wait_add(b):
                pltpu.make_async_copy(mbuf.at[b], accm.at[pl.ds(0, CH)],
                                      asem[b]).wait()

            for b in range(D):
                start_reads(b, b)
            nk = n_sub // D

            def body(kD, carry):
                for b in range(D):
                    jj = D * kD + b
                    wait_reads(b)
                    start_add(b)
                    nxt = jj + D

                    @pl.when(nxt <= n_sub - 1)
                    def _():
                        wait_add(b)
                        start_reads(b, nxt)
                return carry

            lax.fori_loop(0, nk, body, 0)
            for jj in range(D * nk, n_sub):
                b = jj % D
                wait_reads(b)
                start_add(b)
            for b in range(D):
                wait_add(b)

        def dump(out_hbm):
            out_r0 = pl.multiple_of(c * n + r0, 8)
            pltpu.sync_copy(accm.at[pl.ds(r0, s0)],
                            out_hbm.at[pl.ds(out_r0, s0)])
            if tail:
                @pl.when(s == NS - 1)
                def _():
                    t0 = pl.multiple_of(c * n + NS * s0, 8)
                    pltpu.sync_copy(accm.at[pl.ds(NS * s0, tail)],
                                    out_hbm.at[pl.ds(t0, tail)])

        zero_acc()
        plsc.subcore_barrier()
        scatter_phase(m_hbm)
        plsc.subcore_barrier()
        dump(am_out)
        zero_acc()
        plsc.subcore_barrier()
        scatter_phase(tr_hbm)
        plsc.subcore_barrier()
        dump(ax_out)

    return k(m, tr, row, zeros_m)


def _node_call(hf, am, x, ax, mask128, w5h, w5a, b5, w6, b6, blk,
               x_in=None):
    n, H = hf.shape
    final = x_in is not None
    nb = n // blk

    def body(*refs):
        if final:
            (hf_ref, am0_ref, am1_ref, x_ref, ax0_ref, ax1_ref, mk_ref,
             w5h_ref, w5a_ref, b5_ref, w6_ref, b6_ref, xin_ref,
             ho_ref, xo_ref, v_ref) = refs
        else:
            (hf_ref, am0_ref, am1_ref, x_ref, ax0_ref, ax1_ref, mk_ref,
             w5h_ref, w5a_ref, b5_ref, w6_ref, b6_ref,
             ho_ref, xo_ref) = refs
        hv = hf_ref[...]
        agg = am0_ref[...] + am1_ref[...]
        hn = _silu(_dot(hv, w5h_ref[...]) + _dot(agg, w5a_ref[...])
                   + b5_ref[...])
        hn = _dot(hn, w6_ref[...]) + b6_ref[...]
        ho_ref[...] = hv + hn
        mk = mk_ref[...]
        xnew = x_ref[...] + (ax0_ref[...] + ax1_ref[...]) * mk
        xo_ref[...] = xnew
        if final:
            v_ref[...] = (xnew - xin_ref[...]) * mk

    in_specs = [
        pl.BlockSpec((blk, H), lambda i: (i, 0)),
        pl.BlockSpec((blk, H), lambda i: (i, 0)),
        pl.BlockSpec((blk, H), lambda i: (i + nb, 0)),
        pl.BlockSpec((blk, H), lambda i: (i, 0)),
        pl.BlockSpec((blk, H), lambda i: (i, 0)),
        pl.BlockSpec((blk, H), lambda i: (i + nb, 0)),
        pl.BlockSpec((blk, H), lambda i: (i, 0)),
        pl.BlockSpec((H, H), lambda i: (0, 0)),
        pl.BlockSpec((H, H), lambda i: (0, 0)),
        pl.BlockSpec((1, H), lambda i: (0, 0)),
        pl.BlockSpec((H, H), lambda i: (0, 0)),
        pl.BlockSpec((1, H), lambda i: (0, 0)),
    ]
    out_specs = [
        pl.BlockSpec((blk, H), lambda i: (i, 0)),
        pl.BlockSpec((blk, H), lambda i: (i, 0)),
    ]
    out_shape = [
        jax.ShapeDtypeStruct((n, H), F32),
        jax.ShapeDtypeStruct((n, H), F32),
    ]
    args = [hf, am, am, x, ax, ax, mask128, w5h, w5a, b5, w6, b6]
    if final:
        in_specs.append(pl.BlockSpec((blk, H), lambda i: (i, 0)))
        out_specs.append(pl.BlockSpec((blk, H), lambda i: (i, 0)))
        out_shape.append(jax.ShapeDtypeStruct((n, H), F32))
        args.append(x_in)

    return pl.pallas_call(
        body,
        grid=(nb,),
        in_specs=in_specs,
        out_specs=out_specs,
        out_shape=out_shape,
    )(*args)


def kernel(h, x, edge_index, edge_attr, t, mask, params):
    n, node_in = h.shape
    H = params["node_embed"]["W"].shape[1]
    num_layers = len(params["layers"])
    blk = 1000
    eblk = 2000

    row = edge_index[0]
    col = edge_index[1]
    x128 = jnp.pad(x, ((0, 0), (0, H - x.shape[1])))
    mask2 = mask if mask.ndim == 2 else mask[:, None]
    mask128 = jnp.broadcast_to(mask2, (n, H)).astype(F32)

    tp = params["time_mlp"]
    wt1 = tp[0]["W"].reshape(1, H)
    bt1 = tp[0]["b"].reshape(1, H)
    wt2 = tp[1]["W"]
    bt2 = tp[1]["b"].reshape(1, H)
    we = params["edge_embed"]["W"]
    be = params["edge_embed"]["b"].reshape(1, H)
    wn = params["node_embed"]["W"]
    bn = params["node_embed"]["b"].reshape(1, H)

    w1_stack = jnp.stack([lp["edge_mlp"][0]["W"] for lp in params["layers"]])
    b1_stack = jnp.stack([lp["edge_mlp"][0]["b"].reshape(1, H)
                          for lp in params["layers"]])
    w1d_stack = w1_stack[:, 2 * H + 1:, :]

    te, m_fold_stack, beff_stack = _prep_call(
        t, wt1, bt1, wt2, bt2, we, w1d_stack, be, b1_stack, num_layers)

    hf = _embed_call(h, wn, bn, te, blk)

    zeros_m = jnp.zeros((n, H), F32)

    x_in = x128
    v = None
    for l, lp in enumerate(params["layers"]):
        w1 = lp["edge_mlp"][0]["W"]
        w1a = w1[:H, :]
        w1b = w1[H:2 * H, :]
        w1c = w1[2 * H:2 * H + 1, :]
        w2 = lp["edge_mlp"][1]["W"]
        b2 = lp["edge_mlp"][1]["b"].reshape(1, H)
        w3 = lp["coord_mlp"][0]["W"]
        b3 = lp["coord_mlp"][0]["b"].reshape(1, H)
        w4p = jnp.pad(lp["coord_mlp"][1]["W"], ((0, 0), (0, 7)))
        w5 = lp["node_mlp"][0]["W"]
        w5h = w5[:H, :]
        w5a = w5[H:, :]
        b5 = lp["node_mlp"][0]["b"].reshape(1, H)
        w6 = lp["node_mlp"][1]["W"]
        b6 = lp["node_mlp"][1]["b"].reshape(1, H)

        P, Q = _proj_call(hf, w1a, w1b, blk)
        pg, qg, df = _sc_gather_call(P, Q, x128, row, col)
        m, tr = _edge_call(pg, qg, df, edge_attr,
                           m_fold_stack[l], w1c, beff_stack[l],
                           w2, b2, w3, b3, w4p, eblk)
        am, ax = _sc_scatter_call(m, tr, row, zeros_m)
        last = l == num_layers - 1
        outs = _node_call(hf, am, x128, ax, mask128, w5h, w5a, b5, w6, b6,
                          blk, x_in=x_in if last else None)
        if last:
            hf, x128, v = outs
        else:
            hf, x128 = outs

    return v[:, :3]


# speedup vs baseline: 4.2192x; 1.0334x over previous
import functools

import jax
import jax.numpy as jnp
from jax import lax
from jax.experimental import pallas as pl
from jax.experimental.pallas import tpu as pltpu
from jax.experimental.pallas import tpu_sc as plsc

F32 = jnp.float32

NC = 2
NS = 16
NW = NC * NS


def _silu(x):
    return x * jax.nn.sigmoid(x)


def _dot(a, b):
    return jnp.dot(a, b, preferred_element_type=F32)


def _prep_call(t, wt1, bt1, wt2, bt2, we, w1d_stack, be, b1_stack, num_layers):
    def body(t_ref, wt1_ref, bt1_ref, wt2_ref, bt2_ref, we_ref, w1d_ref,
             be_ref, b1_ref, te_ref, m_ref, beff_ref):
        tval = t_ref[0]
        u = _silu(tval * wt1_ref[...] + bt1_ref[...])
        te_ref[...] = _dot(u, wt2_ref[...]) + bt2_ref[...]
        for l in range(num_layers):
            w1d = w1d_ref[l]
            m_ref[l] = _dot(we_ref[...], w1d)
            beff_ref[l] = _dot(be_ref[...], w1d) + b1_ref[l]

    H = wt2.shape[0]
    return pl.pallas_call(
        body,
        in_specs=[
            pl.BlockSpec(memory_space=pltpu.SMEM),
            pl.BlockSpec((1, H), lambda: (0, 0)),
            pl.BlockSpec((1, H), lambda: (0, 0)),
            pl.BlockSpec((H, H), lambda: (0, 0)),
            pl.BlockSpec((1, H), lambda: (0, 0)),
            pl.BlockSpec((16, H), lambda: (0, 0)),
            pl.BlockSpec((num_layers, H, H), lambda: (0, 0, 0)),
            pl.BlockSpec((1, H), lambda: (0, 0)),
            pl.BlockSpec((num_layers, 1, H), lambda: (0, 0, 0)),
        ],
        out_specs=[
            pl.BlockSpec((1, H), lambda: (0, 0)),
            pl.BlockSpec((num_layers, 16, H), lambda: (0, 0, 0)),
            pl.BlockSpec((num_layers, 1, H), lambda: (0, 0, 0)),
        ],
        out_shape=[
            jax.ShapeDtypeStruct((1, H), F32),
            jax.ShapeDtypeStruct((num_layers, 16, H), F32),
            jax.ShapeDtypeStruct((num_layers, 1, H), F32),
        ],
    )(t, wt1, bt1, wt2, bt2, we, w1d_stack, be, b1_stack)


def _embed_call(h, wn, bn, te, blk):
    n, din = h.shape
    H = wn.shape[1]

    def body(h_ref, wn_ref, bn_ref, te_ref, out_ref):
        out_ref[...] = (_dot(h_ref[...], wn_ref[...]) + bn_ref[...]
                        + te_ref[...])

    return pl.pallas_call(
        body,
        grid=(n // blk,),
        in_specs=[
            pl.BlockSpec((blk, din), lambda i: (i, 0)),
            pl.BlockSpec((din, H), lambda i: (0, 0)),
            pl.BlockSpec((1, H), lambda i: (0, 0)),
            pl.BlockSpec((1, H), lambda i: (0, 0)),
        ],
        out_specs=pl.BlockSpec((blk, H), lambda i: (i, 0)),
        out_shape=jax.ShapeDtypeStruct((n, H), F32),
    )(h, wn, bn, te)


def _proj_call(hf, w1a, w1b, blk):
    n, H = hf.shape

    def body(h_ref, wa_ref, wb_ref, p_ref, q_ref):
        hv = h_ref[...]
        p_ref[...] = _dot(hv, wa_ref[...])
        q_ref[...] = _dot(hv, wb_ref[...])

    return pl.pallas_call(
        body,
        grid=(n // blk,),
        in_specs=[
            pl.BlockSpec((blk, H), lambda i: (i, 0)),
            pl.BlockSpec((H, H), lambda i: (0, 0)),
            pl.BlockSpec((H, H), lambda i: (0, 0)),
        ],
        out_specs=[
            pl.BlockSpec((blk, H), lambda i: (i, 0)),
            pl.BlockSpec((blk, H), lambda i: (i, 0)),
        ],
        out_shape=[
            jax.ShapeDtypeStruct((n, H), F32),
            jax.ShapeDtypeStruct((n, H), F32),
        ],
    )(hf, w1a, w1b)


def _sc_gather_call(P, Q, X, row, col, CH):
    n, H = P.shape
    e = row.shape[0]
    e_per_w = e // NW
    n_sub = e_per_w // CH
    mesh = plsc.VectorSubcoreMesh(core_axis_name="c", subcore_axis_name="s")

    D = 2

    @functools.partial(
        pl.kernel,
        out_type=(
            jax.ShapeDtypeStruct((e, H), F32),
            jax.ShapeDtypeStruct((e, H), F32),
            jax.ShapeDtypeStruct((e, H), F32),
        ),
        mesh=mesh,
        scratch_types=[
            pltpu.VMEM((D, CH), jnp.int32),
            pltpu.VMEM((D, CH), jnp.int32),
            pltpu.VMEM((D, CH, H), F32),
            pltpu.VMEM((D, CH, H), F32),
            pltpu.VMEM((D, CH, H), F32),
            pltpu.VMEM((D, CH, H), F32),
            pltpu.VMEM((D, CH, H), F32),
        ] + [pltpu.SemaphoreType.DMA] * (3 * D),
    )
    def k(p_hbm, q_hbm, x_hbm, row_hbm, col_hbm,
          pg_out, qg_out, df_out,
          ridx, cidx, pbuf, qbuf, xrbuf, xcbuf, dfbuf, *sems):
        c = lax.axis_index("c")
        s = lax.axis_index("s")
        wid = s * NC + c
        base = wid * e_per_w
        isem = sems[0:D]
        gsem = sems[D:2 * D]
        wsem = sems[2 * D:3 * D]
        bufs = (pbuf, qbuf, dfbuf)
        outs = (pg_out, qg_out, df_out)

        zeros16 = jnp.zeros((16,), F32)

        def zrow(i, carry):
            for b in range(D):
                for u in range(H // 16):
                    dfbuf[b, i, pl.ds(16 * u, 16)] = zeros16
            return carry

        lax.fori_loop(0, CH, zrow, 0)

        def off_at(jj):
            return pl.multiple_of(base + jj * CH, 8)

        def start_idx(b, jj):
            off = off_at(jj)
            pltpu.async_copy(row_hbm.at[pl.ds(off, CH)], ridx.at[b], isem[b])
            pltpu.async_copy(col_hbm.at[pl.ds(off, CH)], cidx.at[b], isem[b])

        def wait_idx(b):
            for _ in range(2):
                pltpu.make_async_copy(row_hbm.at[pl.ds(0, CH)],
                                      ridx.at[b], isem[b]).wait()

        def start_gathers(b):
            pltpu.async_copy(p_hbm.at[ridx.at[b]], pbuf.at[b], gsem[b])
            pltpu.async_copy(q_hbm.at[cidx.at[b]], qbuf.at[b], gsem[b])
            pltpu.async_copy(x_hbm.at[ridx.at[b]], xrbuf.at[b], gsem[b])
            pltpu.async_copy(x_hbm.at[cidx.at[b]], xcbuf.at[b], gsem[b])

        def wait_gathers(b):
            for _ in range(4):
                pltpu.make_async_copy(p_hbm.at[pl.ds(0, CH)],
                                      pbuf.at[b], gsem[b]).wait()

        def compute_diff(b):
            def sub_row(r, carry):
                dfbuf[b, r, pl.ds(0, 16)] = (xrbuf[b, r, pl.ds(0, 16)]
                                             - xcbuf[b, r, pl.ds(0, 16)])
                return carry
            lax.fori_loop(0, CH, sub_row, 0)

        def start_writes(b, jj):
            off = off_at(jj)
            for buf, out in zip(bufs, outs):
                pltpu.async_copy(buf.at[b], out.at[pl.ds(off, CH)], wsem[b])

        def wait_writes(b):
            for _ in range(3):
                pltpu.make_async_copy(pbuf.at[b], pg_out.at[pl.ds(0, CH)],
                                      wsem[b]).wait()

        for b in range(D):
            start_idx(b, b)
        for b in range(D):
            wait_idx(b)
            start_gathers(b)

        nk = n_sub // D

        def body(kD, carry):
            for b in range(D):
                jj = D * kD + b
                wait_gathers(b)
                compute_diff(b)
                start_writes(b, jj)
                nxt = jj + D

                @pl.when(nxt <= n_sub - 1)
                def _():
                    start_idx(b, nxt)
                    wait_idx(b)
                    wait_writes(b)
                    start_gathers(b)
            return carry

        lax.fori_loop(0, nk, body, 0)
        for jj in range(D * nk, n_sub):
            b = jj % D
            wait_gathers(b)
            compute_diff(b)
            start_writes(b, jj)
        for b in range(D):
            wait_writes(b)

    return k(P, Q, X, row, col)


def _edge_call(pg, qg, df, ea, m_fold, w1c, beff, w2, b2, w3, b3, w4p, blk):
    e, H = pg.shape
    ein = ea.shape[1]

    def body(pg_ref, qg_ref, df_ref, ea_ref, mf_ref, w1c_ref,
             beff_ref, w2_ref, b2_ref, w3_ref, b3_ref, w4_ref,
             m_ref, tr_ref):
        d = df_ref[...]
        dsq = jnp.sum(d * d, axis=1, keepdims=True)
        pre = (pg_ref[...] + qg_ref[...]
               + _dot(ea_ref[...], mf_ref[...])
               + dsq * w1c_ref[...] + beff_ref[...])
        u = _silu(pre)
        m = _silu(_dot(u, w2_ref[...]) + b2_ref[...])
        cw = _silu(_dot(m, w3_ref[...]) + b3_ref[...])
        ws = jnp.tanh(_dot(cw, w4_ref[...]))[:, 0:1]
        dist = jnp.sqrt(dsq + 1e-8)
        tr_ref[...] = d * (ws / (dist + 1e-8))
        m_ref[...] = m

    return pl.pallas_call(
        body,
        grid=(e // blk,),
        in_specs=[
            pl.BlockSpec((blk, H), lambda i: (i, 0)),
            pl.BlockSpec((blk, H), lambda i: (i, 0)),
            pl.BlockSpec((blk, H), lambda i: (i, 0)),
            pl.BlockSpec((blk, ein), lambda i: (i, 0)),
            pl.BlockSpec((ein, H), lambda i: (0, 0)),
            pl.BlockSpec((1, H), lambda i: (0, 0)),
            pl.BlockSpec((1, H), lambda i: (0, 0)),
            pl.BlockSpec((H, H), lambda i: (0, 0)),
            pl.BlockSpec((1, H), lambda i: (0, 0)),
            pl.BlockSpec((H, H), lambda i: (0, 0)),
            pl.BlockSpec((1, H), lambda i: (0, 0)),
            pl.BlockSpec((H, 8), lambda i: (0, 0)),
        ],
        out_specs=[
            pl.BlockSpec((blk, H), lambda i: (i, 0)),
            pl.BlockSpec((blk, H), lambda i: (i, 0)),
        ],
        out_shape=[
            jax.ShapeDtypeStruct((e, H), F32),
            jax.ShapeDtypeStruct((e, H), F32),
        ],
    )(pg, qg, df, ea, m_fold, w1c, beff, w2, b2, w3, b3, w4p)


def _sc_scatter_call(m, tr, row, zeros_m, CH):
    e, H = m.shape
    n = zeros_m.shape[0]
    e_per_w = e // NW
    n_sub = e_per_w // CH
    mesh = plsc.VectorSubcoreMesh(core_axis_name="c", subcore_axis_name="s")

    @functools.partial(
        pl.kernel,
        out_type=(
            jax.ShapeDtypeStruct((NC * n, H), F32),
            jax.ShapeDtypeStruct((NC * n, H), F32),
        ),
        mesh=mesh,
        scratch_types=[
            pltpu.VMEM((4, CH), jnp.int32),
            pltpu.VMEM((4, CH, H), F32),
            pltpu.VMEM_SHARED((n, H), F32),
        ] + [pltpu.SemaphoreType.DMA] * 8,
    )
    def k(m_hbm, tr_hbm, row_hbm, zm_hbm,
          am_out, ax_out, ridx, mbuf, accm, *sems):
        D = 4
        c = lax.axis_index("c")
        s = lax.axis_index("s")
        wid = s * NC + c
        s0 = (n // NS) & ~7
        tail = n - NS * s0
        r0 = pl.multiple_of(s * s0, 8)
        rsem = sems[0:D]
        asem = sems[D:2 * D]

        def zero_acc():
            pltpu.sync_copy(zm_hbm.at[pl.ds(r0, s0)], accm.at[pl.ds(r0, s0)])
            if tail:
                @pl.when(s == NS - 1)
                def _():
                    pltpu.sync_copy(zm_hbm.at[pl.ds(NS * s0, tail)],
                                    accm.at[pl.ds(NS * s0, tail)])

        def scatter_phase(src_hbm):
            def start_reads(b, jj):
                off = pl.multiple_of(wid * e_per_w + jj * CH, 8)
                pltpu.async_copy(row_hbm.at[pl.ds(off, CH)],
                                 ridx.at[b], rsem[b])
                pltpu.async_copy(src_hbm.at[pl.ds(off, CH)],
                                 mbuf.at[b], rsem[b])

            def wait_reads(b):
                pltpu.make_async_copy(row_hbm.at[pl.ds(0, CH)],
                                      ridx.at[b], rsem[b]).wait()
                pltpu.make_async_copy(src_hbm.at[pl.ds(0, CH)],
                                      mbuf.at[b], rsem[b]).wait()

            def start_add(b):
                pltpu.async_copy(mbuf.at[b], accm.at[ridx.at[b]],
                                 asem[b], add=True)

            def wait_add(b):
                pltpu.make_async_copy(mbuf.at[b], accm.at[pl.ds(0, CH)],
                                      asem[b]).wait()

            for b in range(D):
                start_reads(b, b)
            nk = n_sub // D

            def body(kD, carry):
                for b in range(D):
                    jj = D * kD + b
                    wait_reads(b)
                    start_add(b)
                    nxt = jj + D

                    @pl.when(nxt <= n_sub - 1)
                    def _():
                        wait_add(b)
                        start_reads(b, nxt)
                return carry

            lax.fori_loop(0, nk, body, 0)
            for jj in range(D * nk, n_sub):
                b = jj % D
                wait_reads(b)
                start_add(b)
            for b in range(D):
                wait_add(b)

        def dump(out_hbm):
            out_r0 = pl.multiple_of(c * n + r0, 8)
            pltpu.sync_copy(accm.at[pl.ds(r0, s0)],
                            out_hbm.at[pl.ds(out_r0, s0)])
            if tail:
                @pl.when(s == NS - 1)
                def _():
                    t0 = pl.multiple_of(c * n + NS * s0, 8)
                    pltpu.sync_copy(accm.at[pl.ds(NS * s0, tail)],
                                    out_hbm.at[pl.ds(t0, tail)])

        zero_acc()
        plsc.subcore_barrier()
        scatter_phase(m_hbm)
        plsc.subcore_barrier()
        dump(am_out)
        zero_acc()
        plsc.subcore_barrier()
        scatter_phase(tr_hbm)
        plsc.subcore_barrier()
        dump(ax_out)

    return k(m, tr, row, zeros_m)


def _node_call(hf, ams, x, axs, mask128, w5h, w5a, b5, w6, b6, blk,
               x_in=None):
    n, H = hf.shape
    final = x_in is not None
    nb = n // blk
    na = len(ams)

    def body(*refs):
        i = 0
        hf_ref = refs[i]; i += 1
        am_refs = refs[i:i + 2 * na]; i += 2 * na
        x_ref = refs[i]; i += 1
        ax_refs = refs[i:i + 2 * na]; i += 2 * na
        (mk_ref, w5h_ref, w5a_ref, b5_ref, w6_ref, b6_ref) = refs[i:i + 6]
        i += 6
        if final:
            xin_ref = refs[i]; i += 1
        ho_ref = refs[i]
        xo_ref = refs[i + 1]
        hv = hf_ref[...]
        agg = am_refs[0][...]
        for r in am_refs[1:]:
            agg = agg + r[...]
        hn = _silu(_dot(hv, w5h_ref[...]) + _dot(agg, w5a_ref[...])
                   + b5_ref[...])
        hn = _dot(hn, w6_ref[...]) + b6_ref[...]
        ho_ref[...] = hv + hn
        mk = mk_ref[...]
        axsum = ax_refs[0][...]
        for r in ax_refs[1:]:
            axsum = axsum + r[...]
        xnew = x_ref[...] + axsum * mk
        xo_ref[...] = xnew
        if final:
            v_ref = refs[i + 2]
            v_ref[...] = (xnew - xin_ref[...]) * mk

    def part_specs():
        return [pl.BlockSpec((blk, H), lambda i, h=half: (i + h * nb, 0))
                for _ in range(na) for half in (0, 1)]

    in_specs = ([pl.BlockSpec((blk, H), lambda i: (i, 0))]
                + part_specs()
                + [pl.BlockSpec((blk, H), lambda i: (i, 0))]
                + part_specs()
                + [
        pl.BlockSpec((blk, H), lambda i: (i, 0)),
        pl.BlockSpec((H, H), lambda i: (0, 0)),
        pl.BlockSpec((H, H), lambda i: (0, 0)),
        pl.BlockSpec((1, H), lambda i: (0, 0)),
        pl.BlockSpec((H, H), lambda i: (0, 0)),
        pl.BlockSpec((1, H), lambda i: (0, 0)),
    ])
    out_specs = [
        pl.BlockSpec((blk, H), lambda i: (i, 0)),
        pl.BlockSpec((blk, H), lambda i: (i, 0)),
    ]
    out_shape = [
        jax.ShapeDtypeStruct((n, H), F32),
        jax.ShapeDtypeStruct((n, H), F32),
    ]
    am_args = [a for a in ams for _ in range(2)]
    ax_args = [a for a in axs for _ in range(2)]
    args = ([hf] + am_args + [x] + ax_args
            + [mask128, w5h, w5a, b5, w6, b6])
    if final:
        in_specs.append(pl.BlockSpec((blk, H), lambda i: (i, 0)))
        out_specs.append(pl.BlockSpec((blk, H), lambda i: (i, 0)))
        out_shape.append(jax.ShapeDtypeStruct((n, H), F32))
        args.append(x_in)

    return pl.pallas_call(
        body,
        grid=(nb,),
        in_specs=in_specs,
        out_specs=out_specs,
        out_shape=out_shape,
    )(*args)


def kernel(h, x, edge_index, edge_attr, t, mask, params):
    n, node_in = h.shape
    H = params["node_embed"]["W"].shape[1]
    num_layers = len(params["layers"])
    blk = 1000
    eblk = 2000

    row = edge_index[0]
    col = edge_index[1]
    e = row.shape[0]
    nsplit = 2 if (e // 2) % (NW * 8) == 0 else 1
    e_sp = e // nsplit
    ch = 80
    while (e_sp // NW) % ch:
        ch -= 8
    rows = [row[i * e_sp:(i + 1) * e_sp] for i in range(nsplit)]
    cols = [col[i * e_sp:(i + 1) * e_sp] for i in range(nsplit)]
    eas = [edge_attr[i * e_sp:(i + 1) * e_sp] for i in range(nsplit)]
    x128 = jnp.pad(x, ((0, 0), (0, H - x.shape[1])))
    mask2 = mask if mask.ndim == 2 else mask[:, None]
    mask128 = jnp.broadcast_to(mask2, (n, H)).astype(F32)

    tp = params["time_mlp"]
    wt1 = tp[0]["W"].reshape(1, H)
    bt1 = tp[0]["b"].reshape(1, H)
    wt2 = tp[1]["W"]
    bt2 = tp[1]["b"].reshape(1, H)
    we = params["edge_embed"]["W"]
    be = params["edge_embed"]["b"].reshape(1, H)
    wn = params["node_embed"]["W"]
    bn = params["node_embed"]["b"].reshape(1, H)

    w1_stack = jnp.stack([lp["edge_mlp"][0]["W"] for lp in params["layers"]])
    b1_stack = jnp.stack([lp["edge_mlp"][0]["b"].reshape(1, H)
                          for lp in params["layers"]])
    w1d_stack = w1_stack[:, 2 * H + 1:, :]

    te, m_fold_stack, beff_stack = _prep_call(
        t, wt1, bt1, wt2, bt2, we, w1d_stack, be, b1_stack, num_layers)

    hf = _embed_call(h, wn, bn, te, blk)

    zeros_m = jnp.zeros((n, H), F32)

    x_in = x128
    v = None
    for l, lp in enumerate(params["layers"]):
        w1 = lp["edge_mlp"][0]["W"]
        w1a = w1[:H, :]
        w1b = w1[H:2 * H, :]
        w1c = w1[2 * H:2 * H + 1, :]
        w2 = lp["edge_mlp"][1]["W"]
        b2 = lp["edge_mlp"][1]["b"].reshape(1, H)
        w3 = lp["coord_mlp"][0]["W"]
        b3 = lp["coord_mlp"][0]["b"].reshape(1, H)
        w4p = jnp.pad(lp["coord_mlp"][1]["W"], ((0, 0), (0, 7)))
        w5 = lp["node_mlp"][0]["W"]
        w5h = w5[:H, :]
        w5a = w5[H:, :]
        b5 = lp["node_mlp"][0]["b"].reshape(1, H)
        w6 = lp["node_mlp"][1]["W"]
        b6 = lp["node_mlp"][1]["b"].reshape(1, H)

        P, Q = _proj_call(hf, w1a, w1b, blk)
        ams, axs = [], []
        for i in range(nsplit):
            pg, qg, df = _sc_gather_call(P, Q, x128, rows[i], cols[i], ch)
            m, tr = _edge_call(pg, qg, df, eas[i],
                               m_fold_stack[l], w1c, beff_stack[l],
                               w2, b2, w3, b3, w4p, eblk)
            am, ax = _sc_scatter_call(m, tr, rows[i], zeros_m, ch)
            ams.append(am)
            axs.append(ax)
        last = l == num_layers - 1
        outs = _node_call(hf, ams, x128, axs, mask128, w5h, w5a, b5, w6, b6,
                          blk, x_in=x_in if last else None)
        if last:
            hf, x128, v = outs
        else:
            hf, x128 = outs

    return v[:, :3]


# speedup vs baseline: 4.7036x; 1.1148x over previous
import functools

import jax
import jax.numpy as jnp
from jax import lax
from jax.experimental import pallas as pl
from jax.experimental.pallas import tpu as pltpu
from jax.experimental.pallas import tpu_sc as plsc

F32 = jnp.float32

NC = 2
NS = 16
NW = NC * NS


def _silu(x):
    return x * jax.nn.sigmoid(x)


def _dot(a, b):
    return jnp.dot(a, b, preferred_element_type=F32)


def _prep_call(t, wt1, bt1, wt2, bt2, we, w1d_stack, be, b1_stack, num_layers):
    def body(t_ref, wt1_ref, bt1_ref, wt2_ref, bt2_ref, we_ref, w1d_ref,
             be_ref, b1_ref, te_ref, m_ref, beff_ref):
        tval = t_ref[0]
        u = _silu(tval * wt1_ref[...] + bt1_ref[...])
        te_ref[...] = _dot(u, wt2_ref[...]) + bt2_ref[...]
        for l in range(num_layers):
            w1d = w1d_ref[l]
            m_ref[l] = _dot(we_ref[...], w1d)
            beff_ref[l] = _dot(be_ref[...], w1d) + b1_ref[l]

    H = wt2.shape[0]
    return pl.pallas_call(
        body,
        in_specs=[
            pl.BlockSpec(memory_space=pltpu.SMEM),
            pl.BlockSpec((1, H), lambda: (0, 0)),
            pl.BlockSpec((1, H), lambda: (0, 0)),
            pl.BlockSpec((H, H), lambda: (0, 0)),
            pl.BlockSpec((1, H), lambda: (0, 0)),
            pl.BlockSpec((16, H), lambda: (0, 0)),
            pl.BlockSpec((num_layers, H, H), lambda: (0, 0, 0)),
            pl.BlockSpec((1, H), lambda: (0, 0)),
            pl.BlockSpec((num_layers, 1, H), lambda: (0, 0, 0)),
        ],
        out_specs=[
            pl.BlockSpec((1, H), lambda: (0, 0)),
            pl.BlockSpec((num_layers, 16, H), lambda: (0, 0, 0)),
            pl.BlockSpec((num_layers, 1, H), lambda: (0, 0, 0)),
        ],
        out_shape=[
            jax.ShapeDtypeStruct((1, H), F32),
            jax.ShapeDtypeStruct((num_layers, 16, H), F32),
            jax.ShapeDtypeStruct((num_layers, 1, H), F32),
        ],
    )(t, wt1, bt1, wt2, bt2, we, w1d_stack, be, b1_stack)


def _embed_call(h, wn, bn, te, blk):
    n, din = h.shape
    H = wn.shape[1]

    def body(h_ref, wn_ref, bn_ref, te_ref, out_ref):
        out_ref[...] = (_dot(h_ref[...], wn_ref[...]) + bn_ref[...]
                        + te_ref[...])

    return pl.pallas_call(
        body,
        grid=(n // blk,),
        in_specs=[
            pl.BlockSpec((blk, din), lambda i: (i, 0)),
            pl.BlockSpec((din, H), lambda i: (0, 0)),
            pl.BlockSpec((1, H), lambda i: (0, 0)),
            pl.BlockSpec((1, H), lambda i: (0, 0)),
        ],
        out_specs=pl.BlockSpec((blk, H), lambda i: (i, 0)),
        out_shape=jax.ShapeDtypeStruct((n, H), F32),
    )(h, wn, bn, te)


def _proj_call(hf, w1a, w1b, blk):
    n, H = hf.shape

    def body(h_ref, wa_ref, wb_ref, p_ref, q_ref):
        hv = h_ref[...]
        p_ref[...] = _dot(hv, wa_ref[...])
        q_ref[...] = _dot(hv, wb_ref[...])

    return pl.pallas_call(
        body,
        grid=(n // blk,),
        in_specs=[
            pl.BlockSpec((blk, H), lambda i: (i, 0)),
            pl.BlockSpec((H, H), lambda i: (0, 0)),
            pl.BlockSpec((H, H), lambda i: (0, 0)),
        ],
        out_specs=[
            pl.BlockSpec((blk, H), lambda i: (i, 0)),
            pl.BlockSpec((blk, H), lambda i: (i, 0)),
        ],
        out_shape=[
            jax.ShapeDtypeStruct((n, H), F32),
            jax.ShapeDtypeStruct((n, H), F32),
        ],
    )(hf, w1a, w1b)


def _sc_gather_call(P, Q, X, row, col, CH):
    n, H = P.shape
    e = row.shape[0]
    e_per_w = e // NW
    n_sub = e_per_w // CH
    mesh = plsc.VectorSubcoreMesh(core_axis_name="c", subcore_axis_name="s")

    D = 3

    @functools.partial(
        pl.kernel,
        out_type=(
            jax.ShapeDtypeStruct((e, H), F32),
            jax.ShapeDtypeStruct((e, H), F32),
        ),
        mesh=mesh,
        scratch_types=[
            pltpu.VMEM((D, CH), jnp.int32),
            pltpu.VMEM((D, CH), jnp.int32),
            pltpu.VMEM((D, CH, H), F32),
            pltpu.VMEM((D, CH, H), F32),
            pltpu.VMEM((D, CH, H), F32),
            pltpu.VMEM((D, CH, H), F32),
            pltpu.VMEM((D, CH, H), F32),
        ] + [pltpu.SemaphoreType.DMA] * (3 * D),
    )
    def k(p_hbm, q_hbm, x_hbm, row_hbm, col_hbm,
          sg_out, df_out,
          ridx, cidx, pbuf, qbuf, xrbuf, xcbuf, dfbuf, *sems):
        c = lax.axis_index("c")
        s = lax.axis_index("s")
        wid = s * NC + c
        base = wid * e_per_w
        isem = sems[0:D]
        gsem = sems[D:2 * D]
        wsem = sems[2 * D:3 * D]
        bufs = (pbuf, dfbuf)
        outs = (sg_out, df_out)

        zeros16 = jnp.zeros((16,), F32)

        def zrow(i, carry):
            for b in range(D):
                for u in range(H // 16):
                    dfbuf[b, i, pl.ds(16 * u, 16)] = zeros16
            return carry

        lax.fori_loop(0, CH, zrow, 0)

        def off_at(jj):
            return pl.multiple_of(base + jj * CH, 8)

        def start_idx(b, jj):
            off = off_at(jj)
            pltpu.async_copy(row_hbm.at[pl.ds(off, CH)], ridx.at[b], isem[b])
            pltpu.async_copy(col_hbm.at[pl.ds(off, CH)], cidx.at[b], isem[b])

        def wait_idx(b):
            for _ in range(2):
                pltpu.make_async_copy(row_hbm.at[pl.ds(0, CH)],
                                      ridx.at[b], isem[b]).wait()

        def start_gathers(b):
            pltpu.async_copy(p_hbm.at[ridx.at[b]], pbuf.at[b], gsem[b])
            pltpu.async_copy(q_hbm.at[cidx.at[b]], qbuf.at[b], gsem[b])
            pltpu.async_copy(x_hbm.at[ridx.at[b]], xrbuf.at[b], gsem[b])
            pltpu.async_copy(x_hbm.at[cidx.at[b]], xcbuf.at[b], gsem[b])

        def wait_gathers(b):
            for _ in range(4):
                pltpu.make_async_copy(p_hbm.at[pl.ds(0, CH)],
                                      pbuf.at[b], gsem[b]).wait()

        def compute_diff(b):
            def sub_row(r, carry):
                dfbuf[b, r, pl.ds(0, 16)] = (xrbuf[b, r, pl.ds(0, 16)]
                                             - xcbuf[b, r, pl.ds(0, 16)])
                for u in range(H // 16):
                    sl = pl.ds(16 * u, 16)
                    pbuf[b, r, sl] = pbuf[b, r, sl] + qbuf[b, r, sl]
                return carry
            lax.fori_loop(0, CH, sub_row, 0)

        def start_writes(b, jj):
            off = off_at(jj)
            for buf, out in zip(bufs, outs):
                pltpu.async_copy(buf.at[b], out.at[pl.ds(off, CH)], wsem[b])

        def wait_writes(b):
            for _ in range(2):
                pltpu.make_async_copy(pbuf.at[b], sg_out.at[pl.ds(0, CH)],
                                      wsem[b]).wait()

        for b in range(D):
            start_idx(b, b)
        for b in range(D):
            wait_idx(b)
            start_gathers(b)

        nk = n_sub // D

        def body(kD, carry):
            for b in range(D):
                jj = D * kD + b
                wait_gathers(b)
                compute_diff(b)
                start_writes(b, jj)
                nxt = jj + D

                @pl.when(nxt <= n_sub - 1)
                def _():
                    start_idx(b, nxt)
                    wait_idx(b)
                    wait_writes(b)
                    start_gathers(b)
            return carry

        lax.fori_loop(0, nk, body, 0)
        for jj in range(D * nk, n_sub):
            b = jj % D
            wait_gathers(b)
            compute_diff(b)
            start_writes(b, jj)
        for b in range(D):
            wait_writes(b)

    return k(P, Q, X, row, col)


def _edge_call(sg, df, ea, m_fold, w1c, beff, w2, b2, w3, b3, w4p, blk):
    e, H = sg.shape
    ein = ea.shape[1]

    def body(sg_ref, df_ref, ea_ref, mf_ref, w1c_ref,
             beff_ref, w2_ref, b2_ref, w3_ref, b3_ref, w4_ref,
             m_ref, tr_ref):
        d = df_ref[...]
        dsq = jnp.sum(d * d, axis=1, keepdims=True)
        pre = (sg_ref[...]
               + _dot(ea_ref[...], mf_ref[...])
               + dsq * w1c_ref[...] + beff_ref[...])
        u = _silu(pre)
        m = _silu(_dot(u, w2_ref[...]) + b2_ref[...])
        cw = _silu(_dot(m, w3_ref[...]) + b3_ref[...])
        ws = jnp.tanh(_dot(cw, w4_ref[...]))[:, 0:1]
        dist = jnp.sqrt(dsq + 1e-8)
        tr_ref[...] = d * (ws / (dist + 1e-8))
        m_ref[...] = m

    return pl.pallas_call(
        body,
        grid=(e // blk,),
        in_specs=[
            pl.BlockSpec((blk, H), lambda i: (i, 0)),
            pl.BlockSpec((blk, H), lambda i: (i, 0)),
            pl.BlockSpec((blk, ein), lambda i: (i, 0)),
            pl.BlockSpec((ein, H), lambda i: (0, 0)),
            pl.BlockSpec((1, H), lambda i: (0, 0)),
            pl.BlockSpec((1, H), lambda i: (0, 0)),
            pl.BlockSpec((H, H), lambda i: (0, 0)),
            pl.BlockSpec((1, H), lambda i: (0, 0)),
            pl.BlockSpec((H, H), lambda i: (0, 0)),
            pl.BlockSpec((1, H), lambda i: (0, 0)),
            pl.BlockSpec((H, 8), lambda i: (0, 0)),
        ],
        out_specs=[
            pl.BlockSpec((blk, H), lambda i: (i, 0)),
            pl.BlockSpec((blk, H), lambda i: (i, 0)),
        ],
        out_shape=[
            jax.ShapeDtypeStruct((e, H), F32),
            jax.ShapeDtypeStruct((e, H), F32),
        ],
    )(sg, df, ea, m_fold, w1c, beff, w2, b2, w3, b3, w4p)


def _sc_scatter_call(m, tr, row, zeros_m, CH):
    e, H = m.shape
    n = zeros_m.shape[0]
    e_per_w = e // NW
    n_sub = e_per_w // CH
    mesh = plsc.VectorSubcoreMesh(core_axis_name="c", subcore_axis_name="s")

    @functools.partial(
        pl.kernel,
        out_type=(
            jax.ShapeDtypeStruct((NC * n, H), F32),
            jax.ShapeDtypeStruct((NC * n, H), F32),
        ),
        mesh=mesh,
        scratch_types=[
            pltpu.VMEM((4, CH), jnp.int32),
            pltpu.VMEM((4, CH, H), F32),
            pltpu.VMEM_SHARED((n, H), F32),
        ] + [pltpu.SemaphoreType.DMA] * 8,
    )
    def k(m_hbm, tr_hbm, row_hbm, zm_hbm,
          am_out, ax_out, ridx, mbuf, accm, *sems):
        D = 4
        c = lax.axis_index("c")
        s = lax.axis_index("s")
        wid = s * NC + c
        s0 = (n // NS) & ~7
        tail = n - NS * s0
        r0 = pl.multiple_of(s * s0, 8)
        rsem = sems[0:D]
        asem = sems[D:2 * D]

        def zero_acc():
            pltpu.sync_copy(zm_hbm.at[pl.ds(r0, s0)], accm.at[pl.ds(r0, s0)])
            if tail:
                @pl.when(s == NS - 1)
                def _():
                    pltpu.sync_copy(zm_hbm.at[pl.ds(NS * s0, tail)],
                                    accm.at[pl.ds(NS * s0, tail)])

        def scatter_phase(src_hbm):
            def start_reads(b, jj):
                off = pl.multiple_of(wid * e_per_w + jj * CH, 8)
                pltpu.async_copy(row_hbm.at[pl.ds(off, CH)],
                                 ridx.at[b], rsem[b])
                pltpu.async_copy(src_hbm.at[pl.ds(off, CH)],
                                 mbuf.at[b], rsem[b])

            def wait_reads(b):
                pltpu.make_async_copy(row_hbm.at[pl.ds(0, CH)],
                                      ridx.at[b], rsem[b]).wait()
                pltpu.make_async_copy(src_hbm.at[pl.ds(0, CH)],
                                      mbuf.at[b], rsem[b]).wait()

            def start_add(b):
                pltpu.async_copy(mbuf.at[b], accm.at[ridx.at[b]],
                                 asem[b], add=True)

            def wait_add(b):
                pltpu.make_async_copy(mbuf.at[b], accm.at[pl.ds(0, CH)],
                                      asem[b]).wait()

            for b in range(D):
                start_reads(b, b)
            nk = n_sub // D

            def body(kD, carry):
                for b in range(D):
                    jj = D * kD + b
                    wait_reads(b)
                    start_add(b)
                    nxt = jj + D

                    @pl.when(nxt <= n_sub - 1)
                    def _():
                        wait_add(b)
                        start_reads(b, nxt)
                return carry

            lax.fori_loop(0, nk, body, 0)
            for jj in range(D * nk, n_sub):
                b = jj % D
                wait_reads(b)
                start_add(b)
            for b in range(D):
                wait_add(b)

        def dump(out_hbm):
            out_r0 = pl.multiple_of(c * n + r0, 8)
            pltpu.sync_copy(accm.at[pl.ds(r0, s0)],
                            out_hbm.at[pl.ds(out_r0, s0)])
            if tail:
                @pl.when(s == NS - 1)
                def _():
                    t0 = pl.multiple_of(c * n + NS * s0, 8)
                    pltpu.sync_copy(accm.at[pl.ds(NS * s0, tail)],
                                    out_hbm.at[pl.ds(t0, tail)])

        zero_acc()
        plsc.subcore_barrier()
        scatter_phase(m_hbm)
        plsc.subcore_barrier()
        dump(am_out)
        zero_acc()
        plsc.subcore_barrier()
        scatter_phase(tr_hbm)
        plsc.subcore_barrier()
        dump(ax_out)

    return k(m, tr, row, zeros_m)


def _node_call(hf, ams, x, axs, mask128, w5h, w5a, b5, w6, b6, blk,
               x_in=None):
    n, H = hf.shape
    final = x_in is not None
    nb = n // blk
    na = len(ams)

    def body(*refs):
        i = 0
        hf_ref = refs[i]; i += 1
        am_refs = refs[i:i + 2 * na]; i += 2 * na
        x_ref = refs[i]; i += 1
        ax_refs = refs[i:i + 2 * na]; i += 2 * na
        (mk_ref, w5h_ref, w5a_ref, b5_ref, w6_ref, b6_ref) = refs[i:i + 6]
        i += 6
        if final:
            xin_ref = refs[i]; i += 1
        ho_ref = refs[i]
        xo_ref = refs[i + 1]
        hv = hf_ref[...]
        agg = am_refs[0][...]
        for r in am_refs[1:]:
            agg = agg + r[...]
        hn = _silu(_dot(hv, w5h_ref[...]) + _dot(agg, w5a_ref[...])
                   + b5_ref[...])
        hn = _dot(hn, w6_ref[...]) + b6_ref[...]
        ho_ref[...] = hv + hn
        mk = mk_ref[...]
        axsum = ax_refs[0][...]
        for r in ax_refs[1:]:
            axsum = axsum + r[...]
        xnew = x_ref[...] + axsum * mk
        xo_ref[...] = xnew
        if final:
            v_ref = refs[i + 2]
            v_ref[...] = (xnew - xin_ref[...]) * mk

    def part_specs():
        return [pl.BlockSpec((blk, H), lambda i, h=half: (i + h * nb, 0))
                for _ in range(na) for half in (0, 1)]

    in_specs = ([pl.BlockSpec((blk, H), lambda i: (i, 0))]
                + part_specs()
                + [pl.BlockSpec((blk, H), lambda i: (i, 0))]
                + part_specs()
                + [
        pl.BlockSpec((blk, H), lambda i: (i, 0)),
        pl.BlockSpec((H, H), lambda i: (0, 0)),
        pl.BlockSpec((H, H), lambda i: (0, 0)),
        pl.BlockSpec((1, H), lambda i: (0, 0)),
        pl.BlockSpec((H, H), lambda i: (0, 0)),
        pl.BlockSpec((1, H), lambda i: (0, 0)),
    ])
    out_specs = [
        pl.BlockSpec((blk, H), lambda i: (i, 0)),
        pl.BlockSpec((blk, H), lambda i: (i, 0)),
    ]
    out_shape = [
        jax.ShapeDtypeStruct((n, H), F32),
        jax.ShapeDtypeStruct((n, H), F32),
    ]
    am_args = [a for a in ams for _ in range(2)]
    ax_args = [a for a in axs for _ in range(2)]
    args = ([hf] + am_args + [x] + ax_args
            + [mask128, w5h, w5a, b5, w6, b6])
    if final:
        in_specs.append(pl.BlockSpec((blk, H), lambda i: (i, 0)))
        out_specs.append(pl.BlockSpec((blk, H), lambda i: (i, 0)))
        out_shape.append(jax.ShapeDtypeStruct((n, H), F32))
        args.append(x_in)

    return pl.pallas_call(
        body,
        grid=(nb,),
        in_specs=in_specs,
        out_specs=out_specs,
        out_shape=out_shape,
    )(*args)


def kernel(h, x, edge_index, edge_attr, t, mask, params):
    n, node_in = h.shape
    H = params["node_embed"]["W"].shape[1]
    num_layers = len(params["layers"])
    blk = 1000
    eblk = 2000

    row = edge_index[0]
    col = edge_index[1]
    e = row.shape[0]
    nsplit = 2 if (e // 2) % (NW * 8) == 0 else 1
    e_sp = e // nsplit
    ch = 80
    while (e_sp // NW) % ch:
        ch -= 8
    rows = [row[i * e_sp:(i + 1) * e_sp] for i in range(nsplit)]
    cols = [col[i * e_sp:(i + 1) * e_sp] for i in range(nsplit)]
    eas = [edge_attr[i * e_sp:(i + 1) * e_sp] for i in range(nsplit)]
    x128 = jnp.pad(x, ((0, 0), (0, H - x.shape[1])))
    mask2 = mask if mask.ndim == 2 else mask[:, None]
    mask128 = jnp.broadcast_to(mask2, (n, H)).astype(F32)

    tp = params["time_mlp"]
    wt1 = tp[0]["W"].reshape(1, H)
    bt1 = tp[0]["b"].reshape(1, H)
    wt2 = tp[1]["W"]
    bt2 = tp[1]["b"].reshape(1, H)
    we = params["edge_embed"]["W"]
    be = params["edge_embed"]["b"].reshape(1, H)
    wn = params["node_embed"]["W"]
    bn = params["node_embed"]["b"].reshape(1, H)

    w1_stack = jnp.stack([lp["edge_mlp"][0]["W"] for lp in params["layers"]])
    b1_stack = jnp.stack([lp["edge_mlp"][0]["b"].reshape(1, H)
                          for lp in params["layers"]])
    w1d_stack = w1_stack[:, 2 * H + 1:, :]

    te, m_fold_stack, beff_stack = _prep_call(
        t, wt1, bt1, wt2, bt2, we, w1d_stack, be, b1_stack, num_layers)

    hf = _embed_call(h, wn, bn, te, blk)

    zeros_m = jnp.zeros((n, H), F32)

    x_in = x128
    v = None
    for l, lp in enumerate(params["layers"]):
        w1 = lp["edge_mlp"][0]["W"]
        w1a = w1[:H, :]
        w1b = w1[H:2 * H, :]
        w1c = w1[2 * H:2 * H + 1, :]
        w2 = lp["edge_mlp"][1]["W"]
        b2 = lp["edge_mlp"][1]["b"].reshape(1, H)
        w3 = lp["coord_mlp"][0]["W"]
        b3 = lp["coord_mlp"][0]["b"].reshape(1, H)
        w4p = jnp.pad(lp["coord_mlp"][1]["W"], ((0, 0), (0, 7)))
        w5 = lp["node_mlp"][0]["W"]
        w5h = w5[:H, :]
        w5a = w5[H:, :]
        b5 = lp["node_mlp"][0]["b"].reshape(1, H)
        w6 = lp["node_mlp"][1]["W"]
        b6 = lp["node_mlp"][1]["b"].reshape(1, H)

        P, Q = _proj_call(hf, w1a, w1b, blk)
        ams, axs = [], []
        for i in range(nsplit):
            sg, df = _sc_gather_call(P, Q, x128, rows[i], cols[i], ch)
            m, tr = _edge_call(sg, df, eas[i],
                               m_fold_stack[l], w1c, beff_stack[l],
                               w2, b2, w3, b3, w4p, eblk)
            am, ax = _sc_scatter_call(m, tr, rows[i], zeros_m, ch)
            ams.append(am)
            axs.append(ax)
        last = l == num_layers - 1
        outs = _node_call(hf, ams, x128, axs, mask128, w5h, w5a, b5, w6, b6,
                          blk, x_in=x_in if last else None)
        if last:
            hf, x128, v = outs
        else:
            hf, x128 = outs

    return v[:, :3]
